# baseline (speedup 1.0000x reference)
import functools

import jax
import jax.numpy as jnp
from jax import lax
from jax.experimental import pallas as pl
from jax.experimental.pallas import tpu as pltpu
from jax.experimental.pallas import tpu_sc as plsc

N = 5000
EE = 320000
TE = 2 * EE
H = 128
ED = 16
L = 64
NW = 32
CH = 80
EW = TE // NW
NCH = EW // CH
NCHP = 256
EWP = NCHP * CH
ROWS = 10240
RAC = 5120
RW = RAC // 16
EWD = EE // NW
NCHD = EWD // CH
NCHDP = 128

_f32 = jnp.float32
_i32 = jnp.int32


def _proj_body(x_ref, w_ref, b_ref, o_ref):
    y = lax.dot_general(x_ref[0], w_ref[0], (((1,), (1,)), ((), ())),
                        preferred_element_type=_f32) + b_ref[0, 0][None, :]
    o_ref[0] = jnp.where(y > 0, y, jnp.exp(jnp.minimum(y, 0.0)) - 1.0)


def _proj(x_stack, w_stack, b_stack):
    return pl.pallas_call(
        _proj_body,
        grid=(2,),
        in_specs=[
            pl.BlockSpec((1, N, H), lambda d: (d, 0, 0)),
            pl.BlockSpec((1, H, H), lambda d: (d, 0, 0)),
            pl.BlockSpec((1, 1, H), lambda d: (d, 0, 0)),
        ],
        out_specs=pl.BlockSpec((1, N, H), lambda d: (d, 0, 0)),
        out_shape=jax.ShapeDtypeStruct((2, N, H), _f32),
    )(x_stack, w_stack, b_stack)


_EB = 128


def _eatt_body(ea_ref, w_ref, o_ref):
    s = jnp.sum(ea_ref[...] * w_ref[0, 0, 0][None, :], axis=1)
    o_ref[0] = s.reshape(_EB, CH)


def _eatt(ea_pad, w_all):
    nb = NW * NCHP // _EB
    return pl.pallas_call(
        _eatt_body,
        grid=(2, nb),
        in_specs=[
            pl.BlockSpec((_EB * CH, ED), lambda l, i: (i, 0)),
            pl.BlockSpec((1, 1, 1, ED), lambda l, i: (l, i // (nb // 2), 0, 0)),
        ],
        out_specs=pl.BlockSpec((1, _EB, CH), lambda l, i: (l, i, 0)),
        out_shape=jax.ShapeDtypeStruct((2, NW * NCHP, CH), _f32),
    )(ea_pad, w_all)


def _prep_body(xs_ref, xd_ref, w_ref, asrc_ref, wdst_ref, xso_ref, ss_ref, sd_ref):
    xs = lax.dot_general(xs_ref[0], w_ref[0], (((1,), (1,)), ((), ())),
                         preferred_element_type=_f32)
    xso_ref[...] = xs
    ss_ref[0, 0] = jnp.sum(xs * asrc_ref[0, 0][None, :], axis=1)
    sd_ref[0, 0] = jnp.sum(xd_ref[0] * wdst_ref[0, 0][None, :], axis=1)


def _prep(x_state, w_src, a_src, w_dst_att):
    return pl.pallas_call(
        _prep_body,
        grid=(2,),
        in_specs=[
            pl.BlockSpec((1, N, H), lambda d: (1 - d, 0, 0)),
            pl.BlockSpec((1, N, H), lambda d: (d, 0, 0)),
            pl.BlockSpec((1, H, H), lambda d: (d, 0, 0)),
            pl.BlockSpec((1, 1, H), lambda d: (d, 0, 0)),
            pl.BlockSpec((1, 1, H), lambda d: (d, 0, 0)),
        ],
        out_specs=[
            pl.BlockSpec((N, H), lambda d: (d, 0)),
            pl.BlockSpec((1, 1, N), lambda d: (d, 0, 0)),
            pl.BlockSpec((1, 1, N), lambda d: (d, 0, 0)),
        ],
        out_shape=[
            jax.ShapeDtypeStruct((2 * N, H), _f32),
            jax.ShapeDtypeStruct((2, 1, N), _f32),
            jax.ShapeDtypeStruct((2, 1, N), _f32),
        ],
    )(x_state, x_state, w_src, a_src, w_dst_att)


def _combine_body(a128_ref, a16_ref, we_ref, xp_ref, den_ref, o_ref):
    a128 = a128_ref[0, :N]
    a16 = a16_ref[0, :N]
    inv = 1.0 / (den_ref[0, 0, :N] + 1e-16)
    y = (a128 + lax.dot_general(a16, we_ref[0], (((1,), (1,)), ((), ())),
                                preferred_element_type=_f32)) * inv[:, None] + xp_ref[0]
    o_ref[0] = jnp.where(y > 0, y, jnp.exp(jnp.minimum(y, 0.0)) - 1.0)


def _combine(acc128, acc16, w_edge, x_state, den):
    return pl.pallas_call(
        _combine_body,
        grid=(2,),
        in_specs=[
            pl.BlockSpec((1, RAC, H), lambda d: (d, 0, 0)),
            pl.BlockSpec((1, RAC, ED), lambda d: (d, 0, 0)),
            pl.BlockSpec((1, H, ED), lambda d: (d, 0, 0)),
            pl.BlockSpec((1, N, H), lambda d: (d, 0, 0)),
            pl.BlockSpec((1, 1, RAC), lambda d: (d, 0, 0)),
        ],
        out_specs=pl.BlockSpec((1, N, H), lambda d: (d, 0, 0)),
        out_shape=jax.ShapeDtypeStruct((2, N, H), _f32),
    )(acc128, acc16, w_edge, x_state, den.reshape(2, 1, RAC))


def _findec_body(x_ref, wf_ref, bf_ref, w1_ref, b1_ref, w2_ref, b2_ref,
                 w1e_ref, b1e_ref, z_ref, xh_ref, he_ref):
    z = lax.dot_general(x_ref[0], wf_ref[0], (((1,), (1,)), ((), ())),
                        preferred_element_type=_f32) + bf_ref[0, 0][None, :]
    z_ref[0] = z
    h = lax.dot_general(z, w1_ref[0], (((1,), (1,)), ((), ())),
                        preferred_element_type=_f32) + b1_ref[0, 0][None, :]
    h = jnp.maximum(h, 0.0)
    xh_ref[0] = lax.dot_general(h, w2_ref[0], (((1,), (1,)), ((), ())),
                                preferred_element_type=_f32) + b2_ref[0, 0][None, :]
    he_ref[0] = lax.dot_general(z, w1e_ref[0], (((1,), (1,)), ((), ())),
                                preferred_element_type=_f32) + b1e_ref[0, 0][None, :]


def _findec(x_state, wf, bf, w1, b1, w2, b2, w1e, b1e):
    return pl.pallas_call(
        _findec_body,
        grid=(2,),
        in_specs=[
            pl.BlockSpec((1, N, H), lambda d: (d, 0, 0)),
            pl.BlockSpec((1, L, H), lambda d: (d, 0, 0)),
            pl.BlockSpec((1, 1, L), lambda d: (d, 0, 0)),
            pl.BlockSpec((1, H, L), lambda d: (d, 0, 0)),
            pl.BlockSpec((1, 1, H), lambda d: (d, 0, 0)),
            pl.BlockSpec((1, H, H), lambda d: (d, 0, 0)),
            pl.BlockSpec((1, 1, H), lambda d: (d, 0, 0)),
            pl.BlockSpec((1, H, L), lambda d: (d, 0, 0)),
            pl.BlockSpec((1, 1, H), lambda d: (d, 0, 0)),
        ],
        out_specs=[
            pl.BlockSpec((1, N, L), lambda d: (d, 0, 0)),
            pl.BlockSpec((1, N, H), lambda d: (d, 0, 0)),
            pl.BlockSpec((1, N, H), lambda d: (d, 0, 0)),
        ],
        out_shape=[
            jax.ShapeDtypeStruct((2, N, L), _f32),
            jax.ShapeDtypeStruct((2, N, H), _f32),
            jax.ShapeDtypeStruct((2, N, H), _f32),
        ],
    )(x_state, wf, bf, w1, b1, w2, b2, w1e, b1e)


_GB = 10000


def _edgemm_body(g_ref, w_ref, b_ref, o_ref):
    o_ref[...] = lax.dot_general(g_ref[...], w_ref[...], (((1,), (1,)), ((), ())),
                                 preferred_element_type=_f32) + b_ref[0][None, :]


def _edgemm(g, w2e, b2e):
    return pl.pallas_call(
        _edgemm_body,
        grid=(EE // _GB,),
        in_specs=[
            pl.BlockSpec((_GB, H), lambda i: (i, 0)),
            pl.BlockSpec((ED, H), lambda i: (0, 0)),
            pl.BlockSpec((1, ED), lambda i: (0, 0)),
        ],
        out_specs=pl.BlockSpec((_GB, ED), lambda i: (i, 0)),
        out_shape=jax.ShapeDtypeStruct((EE, ED), _f32),
    )(g, w2e, b2e)


_sc_params = pltpu.CompilerParams(needs_layout_passes=False, use_tc_tiling_on_sc=False)
_sc_cache = {}


def _get_mesh():
    return plsc.VectorSubcoreMesh(core_axis_name="c", subcore_axis_name="s")


def _sc_pass1_body(src_hbm, dst_hbm, eatt_hbm, ssrc_hbm, sdst_hbm,
                   den_out, num_out,
                   srcb, dstb, eab, ssrcb, sdstb, numb, zb, den_sh, sem):
    c = lax.axis_index("c")
    s = lax.axis_index("s")
    wid = c * 16 + s
    rowbase = wid * NCHP
    pltpu.sync_copy(src_hbm.at[pl.ds(rowbase, NCHP)], srcb)
    pltpu.sync_copy(dst_hbm.at[pl.ds(rowbase, NCHP)], dstb)
    pltpu.sync_copy(eatt_hbm.at[pl.ds(rowbase, NCHP)], eab)
    pltpu.sync_copy(ssrc_hbm, ssrcb)
    pltpu.sync_copy(sdst_hbm, sdstb)
    zeros = jnp.zeros((16,), _f32)
    for j in range(RW // 16):
        zb[pl.ds(j * 16, 16)] = zeros
    pltpu.sync_copy(zb, den_sh.at[pl.ds(s * RW, RW)])
    plsc.subcore_barrier()
    doff = c * RAC

    def chunk(i, carry):
        for v in range(CH // 16):
            sidx = srcb[i, pl.ds(v * 16, 16)]
            didx = dstb[i, pl.ds(v * 16, 16)] + doff
            a = (plsc.load_gather(ssrcb, [sidx])
                 + plsc.load_gather(sdstb, [didx])
                 + eab[i, pl.ds(v * 16, 16)])
            a = jnp.where(a > 0, a, a * 0.01)
            numb[i, pl.ds(v * 16, 16)] = jnp.exp(a)
        pltpu.sync_copy(numb.at[i], den_sh.at[dstb.at[i]], add=True)
        return carry

    lax.fori_loop(0, NCH, chunk, 0, unroll=False)
    pltpu.sync_copy(numb, num_out.at[pl.ds(rowbase, NCHP)])
    plsc.subcore_barrier()
    pltpu.sync_copy(den_sh.at[pl.ds(s * RW, RW)],
                    den_out.at[pl.ds(c * RAC + s * RW, RW)])


def _sc_pass1(*args):
    if 'p1' not in _sc_cache:
        _sc_cache['p1'] = pl.kernel(
            _sc_pass1_body,
            out_type=[
                jax.ShapeDtypeStruct((2 * RAC,), _f32),
                jax.ShapeDtypeStruct((NW * NCHP, CH), _f32),
            ],
            mesh=_get_mesh(),
            compiler_params=_sc_params,
            scratch_types=[
                pltpu.VMEM((NCHP, CH), _i32),
                pltpu.VMEM((NCHP, CH), _i32),
                pltpu.VMEM((NCHP, CH), _f32),
                pltpu.VMEM((ROWS,), _f32),
                pltpu.VMEM((ROWS,), _f32),
                pltpu.VMEM((NCHP, CH), _f32),
                pltpu.VMEM((RW,), _f32),
                pltpu.VMEM_SHARED((RAC,), _f32),
                pltpu.SemaphoreType.DMA,
            ],
        )
    return _sc_cache['p1'](*args)


def _sc_pass2_body(src_hbm, dst_hbm, num_hbm, xs_hbm, ea_hbm,
                   a128_out, a16_out,
                   srcb, dstb, numb, xsg, eag, z128, z16,
                   a128_sh, a16_sh, semx):
    c = lax.axis_index("c")
    s = lax.axis_index("s")
    wid = c * 16 + s
    rowbase = wid * NCHP
    ebase = wid * EWP
    pltpu.sync_copy(src_hbm.at[pl.ds(rowbase, NCHP)], srcb)
    pltpu.sync_copy(num_hbm.at[pl.ds(rowbase, NCHP)], numb)

    zeros = jnp.zeros((16,), _f32)
    for j in range(16):
        for v in range(H // 16):
            z128[j, pl.ds(v * 16, 16)] = zeros
        z16[j] = zeros

    def zloop(j, carry):
        pltpu.sync_copy(z128, a128_sh.at[pl.ds(s * RW + j * 16, 16)])
        pltpu.sync_copy(z16, a16_sh.at[pl.ds(s * RW + j * 16, 16)])
        return carry

    lax.fori_loop(0, RW // 16, zloop, 0, unroll=False)
    plsc.subcore_barrier()

    def chunk(i, carry):
        sl_i = lax.rem(i, 2)

        @pl.when(i < NCH)
        def _():
            pltpu.async_copy(xs_hbm.at[srcb.at[i]],
                             xsg.at[pl.ds(sl_i * CH, CH)], semx.at[sl_i])

        @pl.when(i > 0)
        def _():
            ip = i - 1
            sl_p = lax.rem(ip, 2)

            @pl.when(lax.rem(ip, 8) == 0)
            def _():
                pltpu.sync_copy(dst_hbm.at[pl.ds(rowbase + ip, 8)], dstb)

            pltpu.sync_copy(ea_hbm.at[pl.ds(ebase + ip * CH, CH)], eag)
            pltpu.make_async_copy(xs_hbm.at[srcb.at[ip]],
                                  xsg.at[pl.ds(sl_p * CH, CH)], semx.at[sl_p]).wait()
            bi = jnp.full((16,), 0, _i32) + ip
            co = sl_p * CH
            for j in range(CH):
                bj = jnp.full((16,), j, _i32)
                nv = plsc.load_gather(numb, [bi, bj])
                for k in range(H // 16):
                    xsg[co + j, pl.ds(k * 16, 16)] = xsg[co + j, pl.ds(k * 16, 16)] * nv
                eag[j] = eag[j] * nv
            dr = lax.rem(ip, 8)
            pltpu.sync_copy(xsg.at[pl.ds(sl_p * CH, CH)], a128_sh.at[dstb.at[dr]], add=True)
            pltpu.sync_copy(eag, a16_sh.at[dstb.at[dr]], add=True)

        return carry

    lax.fori_loop(0, NCH + 1, chunk, 0, unroll=False)
    plsc.subcore_barrier()
    pltpu.sync_copy(a128_sh.at[pl.ds(s * RW, RW)], a128_out.at[c, pl.ds(s * RW, RW), :])
    pltpu.sync_copy(a16_sh.at[pl.ds(s * RW, RW)], a16_out.at[c, pl.ds(s * RW, RW), :])


def _sc_pass2(*args):
    if 'p2' not in _sc_cache:
        _sc_cache['p2'] = pl.kernel(
            _sc_pass2_body,
            out_type=[
                jax.ShapeDtypeStruct((2, RAC, H), _f32),
                jax.ShapeDtypeStruct((2, RAC, ED), _f32),
            ],
            mesh=_get_mesh(),
            compiler_params=_sc_params,
            scratch_types=[
                pltpu.VMEM((NCHP, CH), _i32),
                pltpu.VMEM((8, CH), _i32),
                pltpu.VMEM((NCHP, CH), _f32),
                pltpu.VMEM((2 * CH, H), _f32),
                pltpu.VMEM((CH, ED), _f32),
                pltpu.VMEM((16, H), _f32),
                pltpu.VMEM((16, ED), _f32),
                pltpu.VMEM_SHARED((RAC, H), _f32),
                pltpu.VMEM_SHARED((RAC, ED), _f32),
                pltpu.SemaphoreType.DMA((2,)),
            ],
        )
    return _sc_cache['p2'](*args)


def _sc_edge_body(row_hbm, col_hbm, hp_hbm, hm_hbm, g_out,
                  rowb, colb, hpg, hmg, semp, semm):
    c = lax.axis_index("c")
    s = lax.axis_index("s")
    wid = c * 16 + s
    rowbase = wid * NCHDP
    ebase = wid * EWD
    pltpu.sync_copy(row_hbm.at[pl.ds(rowbase, NCHDP)], rowb)
    pltpu.sync_copy(col_hbm.at[pl.ds(rowbase, NCHDP)], colb)

    pltpu.async_copy(hp_hbm.at[rowb.at[0]], hpg.at[pl.ds(0, CH)], semp.at[0])
    pltpu.async_copy(hm_hbm.at[colb.at[0]], hmg.at[pl.ds(0, CH)], semm.at[0])

    def chunk(i, carry):
        cur = lax.rem(i, 2)
        nxt = lax.rem(i + 1, 2)

        @pl.when(i + 1 < NCHD)
        def _():
            pltpu.async_copy(hp_hbm.at[rowb.at[i + 1]],
                             hpg.at[pl.ds(nxt * CH, CH)], semp.at[nxt])
            pltpu.async_copy(hm_hbm.at[colb.at[i + 1]],
                             hmg.at[pl.ds(nxt * CH, CH)], semm.at[nxt])

        pltpu.make_async_copy(hp_hbm.at[rowb.at[i]],
                              hpg.at[pl.ds(cur * CH, CH)], semp.at[cur]).wait()
        pltpu.make_async_copy(hm_hbm.at[colb.at[i]],
                              hmg.at[pl.ds(cur * CH, CH)], semm.at[cur]).wait()
        co = cur * CH
        for j in range(CH):
            for k in range(H // 16):
                sl = pl.ds(k * 16, 16)
                hpg[co + j, sl] = jnp.maximum(hpg[co + j, sl] + hmg[co + j, sl], 0.0)
        pltpu.sync_copy(hpg.at[pl.ds(cur * CH, CH)], g_out.at[pl.ds(ebase + i * CH, CH)])
        return carry

    lax.fori_loop(0, NCHD, chunk, 0, unroll=False)


def _sc_edge(*args):
    if 'pe' not in _sc_cache:
        _sc_cache['pe'] = pl.kernel(
            _sc_edge_body,
            out_type=jax.ShapeDtypeStruct((EE, H), _f32),
            mesh=_get_mesh(),
            compiler_params=_sc_params,
            scratch_types=[
                pltpu.VMEM((NCHDP, CH), _i32),
                pltpu.VMEM((NCHDP, CH), _i32),
                pltpu.VMEM((2 * CH, H), _f32),
                pltpu.VMEM((2 * CH, H), _f32),
                pltpu.SemaphoreType.DMA((2,)),
                pltpu.SemaphoreType.DMA((2,)),
            ],
        )
    return _sc_cache['pe'](*args)


def kernel(x_member, x_provider, edge_index_pm, edge_index_mp, edge_attr_pm, edge_attr_mp, params):
    ei_pm = edge_index_pm.astype(_i32)
    ei_mp = edge_index_mp.astype(_i32)
    def _pad_idx(flat, ew, nchp):
        a = flat.reshape(NW, ew)
        a = jnp.pad(a, ((0, 0), (0, nchp * CH - ew)))
        return a.reshape(NW * nchp, CH)

    src2d = _pad_idx(jnp.concatenate([ei_pm[0], ei_mp[0] + N]), EW, NCHP)
    dst2d = _pad_idx(jnp.concatenate([ei_pm[1], ei_mp[1]]), EW, NCHP)
    row2d = _pad_idx(ei_pm[0], EWD, NCHDP)
    col2d = _pad_idx(ei_pm[1], EWD, NCHDP)
    ea_cat = jnp.concatenate([edge_attr_pm, edge_attr_mp]).reshape(NW, EW, ED)
    ea_pad = jnp.pad(ea_cat, ((0, 0), (0, EWP - EW), (0, 0))).reshape(NW * EWP, ED)

    convs = params['convs']
    w_all = jnp.stack([
        jnp.stack([(cv[d]['W_edge'].T @ cv[d]['att_edge'][0])[None, :]
                   for d in ('pm', 'mp')])
        for cv in convs])
    eatt = _eatt(ea_pad, w_all)

    x_in = jnp.stack([x_member, x_provider])
    wp = jnp.stack([params['proj_member']['W'], params['proj_provider']['W']])
    bp = jnp.stack([params['proj_member']['b'], params['proj_provider']['b']])[:, None, :]
    x_state = _proj(x_in, wp, bp)

    for li, cv in enumerate(convs):
        w_src = jnp.stack([cv['pm']['W_src'], cv['mp']['W_src']])
        a_src = jnp.stack([cv['pm']['att_src'], cv['mp']['att_src']])
        w_dst_att = jnp.stack([(cv['pm']['att_dst'][0] @ cv['pm']['W_dst'])[None, :],
                               (cv['mp']['att_dst'][0] @ cv['mp']['W_dst'])[None, :]])
        w_edge = jnp.stack([cv['pm']['W_edge'], cv['mp']['W_edge']])
        xs_cat, s_src, s_dst = _prep(x_state, w_src, a_src, w_dst_att)
        ss_flat = jnp.pad(s_src.reshape(2 * N), (0, ROWS - 2 * N))
        sd_flat = jnp.pad(s_dst, ((0, 0), (0, 0), (0, RAC - N))).reshape(2 * RAC)
        den, num = _sc_pass1(src2d, dst2d, eatt[li], ss_flat, sd_flat)
        acc128, acc16 = _sc_pass2(src2d, dst2d, num, xs_cat, ea_pad)
        x_state = _combine(acc128, acc16, w_edge, x_state, den)

    wf = jnp.stack([params['final_member']['W'], params['final_provider']['W']])
    bf = jnp.stack([params['final_member']['b'], params['final_provider']['b']])[:, None, :]
    dm, dp, de = params['dec_member'], params['dec_provider'], params['dec_edge']
    w1 = jnp.stack([dm['W1'], dp['W1']])
    b1 = jnp.stack([dm['b1'], dp['b1']])[:, None, :]
    w2 = jnp.stack([dm['W2'], dp['W2']])
    b2 = jnp.stack([dm['b2'], dp['b2']])[:, None, :]
    w1e = jnp.stack([de['W1'][:, L:], de['W1'][:, :L]])
    b1e = jnp.stack([jnp.zeros_like(de['b1']), de['b1']])[:, None, :]
    z, xhat, hedge = _findec(x_state, wf, bf, w1, b1, w2, b2, w1e, b1e)

    g = _sc_edge(row2d, col2d, hedge[1], hedge[0])
    edge_hat = _edgemm(g, de['W2'], de['b2'][None, :])

    return xhat[0], xhat[1], z[0], z[1], edge_hat


# speedup vs baseline: 1.0293x; 1.0293x over previous
import functools

import jax
import jax.numpy as jnp
from jax import lax
from jax.experimental import pallas as pl
from jax.experimental.pallas import tpu as pltpu
from jax.experimental.pallas import tpu_sc as plsc

N = 5000
EE = 320000
TE = 2 * EE
H = 128
ED = 16
L = 64
NW = 32
CH = 80
EW = TE // NW
NCH = EW // CH
NCHP = 256
EWP = NCHP * CH
ROWS = 10240
RAC = 5120
RW = RAC // 16
EWD = EE // NW
NCHD = EWD // CH
NCHDP = 128

_f32 = jnp.float32
_i32 = jnp.int32


def _proj_body(x_ref, w_ref, b_ref, o_ref):
    y = lax.dot_general(x_ref[0], w_ref[0], (((1,), (1,)), ((), ())),
                        preferred_element_type=_f32) + b_ref[0, 0][None, :]
    o_ref[0] = jnp.where(y > 0, y, jnp.exp(jnp.minimum(y, 0.0)) - 1.0)


def _proj(x_stack, w_stack, b_stack):
    return pl.pallas_call(
        _proj_body,
        grid=(2,),
        in_specs=[
            pl.BlockSpec((1, N, H), lambda d: (d, 0, 0)),
            pl.BlockSpec((1, H, H), lambda d: (d, 0, 0)),
            pl.BlockSpec((1, 1, H), lambda d: (d, 0, 0)),
        ],
        out_specs=pl.BlockSpec((1, N, H), lambda d: (d, 0, 0)),
        out_shape=jax.ShapeDtypeStruct((2, N, H), _f32),
    )(x_stack, w_stack, b_stack)


_EB = 128


def _eatt_body(ea_ref, w_ref, o_ref):
    s = jnp.sum(ea_ref[...] * w_ref[0, 0, 0][None, :], axis=1)
    o_ref[0] = s.reshape(_EB, CH)


def _eatt(ea_pad, w_all):
    nb = NW * NCHP // _EB
    return pl.pallas_call(
        _eatt_body,
        grid=(2, nb),
        in_specs=[
            pl.BlockSpec((_EB * CH, ED), lambda l, i: (i, 0)),
            pl.BlockSpec((1, 1, 1, ED), lambda l, i: (l, i // (nb // 2), 0, 0)),
        ],
        out_specs=pl.BlockSpec((1, _EB, CH), lambda l, i: (l, i, 0)),
        out_shape=jax.ShapeDtypeStruct((2, NW * NCHP, CH), _f32),
    )(ea_pad, w_all)


def _prep_body(xs_ref, xd_ref, w_ref, asrc_ref, wdst_ref, xso_ref, ss_ref, sd_ref):
    xs = lax.dot_general(xs_ref[0], w_ref[0], (((1,), (1,)), ((), ())),
                         preferred_element_type=_f32)
    xso_ref[...] = xs
    ss_ref[0, 0] = jnp.sum(xs * asrc_ref[0, 0][None, :], axis=1)
    sd_ref[0, 0] = jnp.sum(xd_ref[0] * wdst_ref[0, 0][None, :], axis=1)


def _prep(x_state, w_src, a_src, w_dst_att):
    return pl.pallas_call(
        _prep_body,
        grid=(2,),
        in_specs=[
            pl.BlockSpec((1, N, H), lambda d: (1 - d, 0, 0)),
            pl.BlockSpec((1, N, H), lambda d: (d, 0, 0)),
            pl.BlockSpec((1, H, H), lambda d: (d, 0, 0)),
            pl.BlockSpec((1, 1, H), lambda d: (d, 0, 0)),
            pl.BlockSpec((1, 1, H), lambda d: (d, 0, 0)),
        ],
        out_specs=[
            pl.BlockSpec((N, H), lambda d: (d, 0)),
            pl.BlockSpec((1, 1, N), lambda d: (d, 0, 0)),
            pl.BlockSpec((1, 1, N), lambda d: (d, 0, 0)),
        ],
        out_shape=[
            jax.ShapeDtypeStruct((2 * N, H), _f32),
            jax.ShapeDtypeStruct((2, 1, N), _f32),
            jax.ShapeDtypeStruct((2, 1, N), _f32),
        ],
    )(x_state, x_state, w_src, a_src, w_dst_att)


def _combine_body(a128_ref, a16_ref, we_ref, xp_ref, den_ref, o_ref):
    a128 = a128_ref[0, :N]
    a16 = a16_ref[0, :N]
    inv = 1.0 / (den_ref[0, 0, :N] + 1e-16)
    y = (a128 + lax.dot_general(a16, we_ref[0], (((1,), (1,)), ((), ())),
                                preferred_element_type=_f32)) * inv[:, None] + xp_ref[0]
    o_ref[0] = jnp.where(y > 0, y, jnp.exp(jnp.minimum(y, 0.0)) - 1.0)


def _combine(acc128, acc16, w_edge, x_state, den):
    return pl.pallas_call(
        _combine_body,
        grid=(2,),
        in_specs=[
            pl.BlockSpec((1, RAC, H), lambda d: (d, 0, 0)),
            pl.BlockSpec((1, RAC, ED), lambda d: (d, 0, 0)),
            pl.BlockSpec((1, H, ED), lambda d: (d, 0, 0)),
            pl.BlockSpec((1, N, H), lambda d: (d, 0, 0)),
            pl.BlockSpec((1, 1, RAC), lambda d: (d, 0, 0)),
        ],
        out_specs=pl.BlockSpec((1, N, H), lambda d: (d, 0, 0)),
        out_shape=jax.ShapeDtypeStruct((2, N, H), _f32),
    )(acc128, acc16, w_edge, x_state, den.reshape(2, 1, RAC))


def _findec_body(x_ref, wf_ref, bf_ref, w1_ref, b1_ref, w2_ref, b2_ref,
                 w1e_ref, b1e_ref, z_ref, xh_ref, he_ref):
    z = lax.dot_general(x_ref[0], wf_ref[0], (((1,), (1,)), ((), ())),
                        preferred_element_type=_f32) + bf_ref[0, 0][None, :]
    z_ref[0] = z
    h = lax.dot_general(z, w1_ref[0], (((1,), (1,)), ((), ())),
                        preferred_element_type=_f32) + b1_ref[0, 0][None, :]
    h = jnp.maximum(h, 0.0)
    xh_ref[0] = lax.dot_general(h, w2_ref[0], (((1,), (1,)), ((), ())),
                                preferred_element_type=_f32) + b2_ref[0, 0][None, :]
    he_ref[0] = lax.dot_general(z, w1e_ref[0], (((1,), (1,)), ((), ())),
                                preferred_element_type=_f32) + b1e_ref[0, 0][None, :]


def _findec(x_state, wf, bf, w1, b1, w2, b2, w1e, b1e):
    return pl.pallas_call(
        _findec_body,
        grid=(2,),
        in_specs=[
            pl.BlockSpec((1, N, H), lambda d: (d, 0, 0)),
            pl.BlockSpec((1, L, H), lambda d: (d, 0, 0)),
            pl.BlockSpec((1, 1, L), lambda d: (d, 0, 0)),
            pl.BlockSpec((1, H, L), lambda d: (d, 0, 0)),
            pl.BlockSpec((1, 1, H), lambda d: (d, 0, 0)),
            pl.BlockSpec((1, H, H), lambda d: (d, 0, 0)),
            pl.BlockSpec((1, 1, H), lambda d: (d, 0, 0)),
            pl.BlockSpec((1, H, L), lambda d: (d, 0, 0)),
            pl.BlockSpec((1, 1, H), lambda d: (d, 0, 0)),
        ],
        out_specs=[
            pl.BlockSpec((1, N, L), lambda d: (d, 0, 0)),
            pl.BlockSpec((1, N, H), lambda d: (d, 0, 0)),
            pl.BlockSpec((1, N, H), lambda d: (d, 0, 0)),
        ],
        out_shape=[
            jax.ShapeDtypeStruct((2, N, L), _f32),
            jax.ShapeDtypeStruct((2, N, H), _f32),
            jax.ShapeDtypeStruct((2, N, H), _f32),
        ],
    )(x_state, wf, bf, w1, b1, w2, b2, w1e, b1e)


_GB = 10000


def _edgemm_body(g_ref, w_ref, b_ref, o_ref):
    o_ref[...] = lax.dot_general(g_ref[...], w_ref[...], (((1,), (1,)), ((), ())),
                                 preferred_element_type=_f32) + b_ref[0][None, :]


def _edgemm(g, w2e, b2e):
    return pl.pallas_call(
        _edgemm_body,
        grid=(EE // _GB,),
        in_specs=[
            pl.BlockSpec((_GB, H), lambda i: (i, 0)),
            pl.BlockSpec((ED, H), lambda i: (0, 0)),
            pl.BlockSpec((1, ED), lambda i: (0, 0)),
        ],
        out_specs=pl.BlockSpec((_GB, ED), lambda i: (i, 0)),
        out_shape=jax.ShapeDtypeStruct((EE, ED), _f32),
    )(g, w2e, b2e)


_sc_params = pltpu.CompilerParams(needs_layout_passes=False, use_tc_tiling_on_sc=False)
_sc_cache = {}


def _get_mesh():
    return plsc.VectorSubcoreMesh(core_axis_name="c", subcore_axis_name="s")


def _sc_pass1_body(src_hbm, dst_hbm, eatt_hbm, ssrc_hbm, sdst_hbm,
                   den_out, num_out,
                   srcb, dstb, eab, ssrcb, sdstb, numb, zb, den_sh, sem):
    c = lax.axis_index("c")
    s = lax.axis_index("s")
    wid = c * 16 + s
    rowbase = wid * NCHP
    pltpu.sync_copy(src_hbm.at[pl.ds(rowbase, NCHP)], srcb)
    pltpu.sync_copy(dst_hbm.at[pl.ds(rowbase, NCHP)], dstb)
    pltpu.sync_copy(eatt_hbm.at[pl.ds(rowbase, NCHP)], eab)
    pltpu.sync_copy(ssrc_hbm, ssrcb)
    pltpu.sync_copy(sdst_hbm, sdstb)
    zeros = jnp.zeros((16,), _f32)
    for j in range(RW // 16):
        zb[pl.ds(j * 16, 16)] = zeros
    pltpu.sync_copy(zb, den_sh.at[pl.ds(s * RW, RW)])
    plsc.subcore_barrier()
    doff = c * RAC

    def chunk(i, carry):
        for v in range(CH // 16):
            sidx = srcb[i, pl.ds(v * 16, 16)]
            didx = dstb[i, pl.ds(v * 16, 16)] + doff
            a = (plsc.load_gather(ssrcb, [sidx])
                 + plsc.load_gather(sdstb, [didx])
                 + eab[i, pl.ds(v * 16, 16)])
            a = jnp.where(a > 0, a, a * 0.01)
            numb[i, pl.ds(v * 16, 16)] = jnp.exp(a)
        pltpu.sync_copy(numb.at[i], den_sh.at[dstb.at[i]], add=True)
        return carry

    lax.fori_loop(0, NCH, chunk, 0, unroll=False)
    pltpu.sync_copy(numb, num_out.at[pl.ds(rowbase, NCHP)])
    plsc.subcore_barrier()
    pltpu.sync_copy(den_sh.at[pl.ds(s * RW, RW)],
                    den_out.at[pl.ds(c * RAC + s * RW, RW)])


def _sc_pass1(*args):
    if 'p1' not in _sc_cache:
        _sc_cache['p1'] = pl.kernel(
            _sc_pass1_body,
            out_type=[
                jax.ShapeDtypeStruct((2 * RAC,), _f32),
                jax.ShapeDtypeStruct((NW * NCHP, CH), _f32),
            ],
            mesh=_get_mesh(),
            compiler_params=_sc_params,
            scratch_types=[
                pltpu.VMEM((NCHP, CH), _i32),
                pltpu.VMEM((NCHP, CH), _i32),
                pltpu.VMEM((NCHP, CH), _f32),
                pltpu.VMEM((ROWS,), _f32),
                pltpu.VMEM((ROWS,), _f32),
                pltpu.VMEM((NCHP, CH), _f32),
                pltpu.VMEM((RW,), _f32),
                pltpu.VMEM_SHARED((RAC,), _f32),
                pltpu.SemaphoreType.DMA,
            ],
        )
    return _sc_cache['p1'](*args)


def _sc_pass2_body(src_hbm, dst_hbm, num_hbm, xs_hbm, ea_hbm,
                   a128_out, a16_out,
                   srcb, dstb, numb, xsg, eag, z128, z16,
                   a128_sh, a16_sh, semx, semsc, semse):
    c = lax.axis_index("c")
    s = lax.axis_index("s")
    wid = c * 16 + s
    rowbase = wid * NCHP
    ebase = wid * EWP
    pltpu.sync_copy(src_hbm.at[pl.ds(rowbase, NCHP)], srcb)
    pltpu.sync_copy(num_hbm.at[pl.ds(rowbase, NCHP)], numb)

    zeros = jnp.zeros((16,), _f32)
    for j in range(16):
        for v in range(H // 16):
            z128[j, pl.ds(v * 16, 16)] = zeros
        z16[j] = zeros

    def zloop(j, carry):
        pltpu.sync_copy(z128, a128_sh.at[pl.ds(s * RW + j * 16, 16)])
        pltpu.sync_copy(z16, a16_sh.at[pl.ds(s * RW + j * 16, 16)])
        return carry

    lax.fori_loop(0, RW // 16, zloop, 0, unroll=False)
    plsc.subcore_barrier()

    def chunk(i, carry):
        sl_i = lax.rem(i, 2)

        @pl.when(i < NCH)
        def _():
            @pl.when(i >= 2)
            def _():
                dr2 = lax.rem(i - 2, 8)
                pltpu.make_async_copy(xsg.at[pl.ds(sl_i * CH, CH)],
                                      a128_sh.at[dstb.at[dr2]], semsc.at[sl_i]).wait()
                pltpu.make_async_copy(eag.at[pl.ds(sl_i * CH, CH)],
                                      a16_sh.at[dstb.at[dr2]], semse.at[sl_i]).wait()

            pltpu.async_copy(xs_hbm.at[srcb.at[i]],
                             xsg.at[pl.ds(sl_i * CH, CH)], semx.at[sl_i])

        @pl.when(i > 0)
        def _():
            ip = i - 1
            sl_p = lax.rem(ip, 2)

            @pl.when(lax.rem(ip, 8) == 0)
            def _():
                pltpu.sync_copy(dst_hbm.at[pl.ds(rowbase + ip, 8)], dstb)

            pltpu.sync_copy(ea_hbm.at[pl.ds(ebase + ip * CH, CH)],
                            eag.at[pl.ds(sl_p * CH, CH)])
            pltpu.make_async_copy(xs_hbm.at[srcb.at[ip]],
                                  xsg.at[pl.ds(sl_p * CH, CH)], semx.at[sl_p]).wait()
            bi = jnp.full((16,), 0, _i32) + ip
            co = sl_p * CH
            for j in range(CH):
                bj = jnp.full((16,), j, _i32)
                nv = plsc.load_gather(numb, [bi, bj])
                for k in range(H // 16):
                    xsg[co + j, pl.ds(k * 16, 16)] = xsg[co + j, pl.ds(k * 16, 16)] * nv
                eag[co + j] = eag[co + j] * nv
            dr = lax.rem(ip, 8)
            pltpu.async_copy(xsg.at[pl.ds(sl_p * CH, CH)],
                             a128_sh.at[dstb.at[dr]], semsc.at[sl_p], add=True)
            pltpu.async_copy(eag.at[pl.ds(sl_p * CH, CH)],
                             a16_sh.at[dstb.at[dr]], semse.at[sl_p], add=True)

        return carry

    lax.fori_loop(0, NCH + 1, chunk, 0, unroll=False)
    for sl in (0, 1):
        dr2 = (NCH - 2 + sl) % 8
        pltpu.make_async_copy(xsg.at[pl.ds(sl * CH, CH)],
                              a128_sh.at[dstb.at[dr2]], semsc.at[sl]).wait()
        pltpu.make_async_copy(eag.at[pl.ds(sl * CH, CH)],
                              a16_sh.at[dstb.at[dr2]], semse.at[sl]).wait()
    plsc.subcore_barrier()
    pltpu.sync_copy(a128_sh.at[pl.ds(s * RW, RW)], a128_out.at[c, pl.ds(s * RW, RW), :])
    pltpu.sync_copy(a16_sh.at[pl.ds(s * RW, RW)], a16_out.at[c, pl.ds(s * RW, RW), :])


def _sc_pass2(*args):
    if 'p2' not in _sc_cache:
        _sc_cache['p2'] = pl.kernel(
            _sc_pass2_body,
            out_type=[
                jax.ShapeDtypeStruct((2, RAC, H), _f32),
                jax.ShapeDtypeStruct((2, RAC, ED), _f32),
            ],
            mesh=_get_mesh(),
            compiler_params=_sc_params,
            scratch_types=[
                pltpu.VMEM((NCHP, CH), _i32),
                pltpu.VMEM((8, CH), _i32),
                pltpu.VMEM((NCHP, CH), _f32),
                pltpu.VMEM((2 * CH, H), _f32),
                pltpu.VMEM((2 * CH, ED), _f32),
                pltpu.VMEM((16, H), _f32),
                pltpu.VMEM((16, ED), _f32),
                pltpu.VMEM_SHARED((RAC, H), _f32),
                pltpu.VMEM_SHARED((RAC, ED), _f32),
                pltpu.SemaphoreType.DMA((2,)),
                pltpu.SemaphoreType.DMA((2,)),
                pltpu.SemaphoreType.DMA((2,)),
            ],
        )
    return _sc_cache['p2'](*args)


def _sc_edge_body(row_hbm, col_hbm, hp_hbm, hm_hbm, g_out,
                  rowb, colb, hpg, hmg, semp, semm, semw):
    c = lax.axis_index("c")
    s = lax.axis_index("s")
    wid = c * 16 + s
    rowbase = wid * NCHDP
    ebase = wid * EWD
    pltpu.sync_copy(row_hbm.at[pl.ds(rowbase, NCHDP)], rowb)
    pltpu.sync_copy(col_hbm.at[pl.ds(rowbase, NCHDP)], colb)

    pltpu.async_copy(hp_hbm.at[rowb.at[0]], hpg.at[pl.ds(0, CH)], semp.at[0])
    pltpu.async_copy(hm_hbm.at[colb.at[0]], hmg.at[pl.ds(0, CH)], semm.at[0])

    def chunk(i, carry):
        cur = lax.rem(i, 2)
        nxt = lax.rem(i + 1, 2)

        @pl.when(i + 1 < NCHD)
        def _():
            @pl.when(i >= 1)
            def _():
                pltpu.make_async_copy(hpg.at[pl.ds(nxt * CH, CH)],
                                      g_out.at[pl.ds(ebase + (i - 1) * CH, CH)],
                                      semw.at[nxt]).wait()

            pltpu.async_copy(hp_hbm.at[rowb.at[i + 1]],
                             hpg.at[pl.ds(nxt * CH, CH)], semp.at[nxt])
            pltpu.async_copy(hm_hbm.at[colb.at[i + 1]],
                             hmg.at[pl.ds(nxt * CH, CH)], semm.at[nxt])

        pltpu.make_async_copy(hp_hbm.at[rowb.at[i]],
                              hpg.at[pl.ds(cur * CH, CH)], semp.at[cur]).wait()
        pltpu.make_async_copy(hm_hbm.at[colb.at[i]],
                              hmg.at[pl.ds(cur * CH, CH)], semm.at[cur]).wait()
        co = cur * CH
        for j in range(CH):
            for k in range(H // 16):
                sl = pl.ds(k * 16, 16)
                hpg[co + j, sl] = jnp.maximum(hpg[co + j, sl] + hmg[co + j, sl], 0.0)
        pltpu.async_copy(hpg.at[pl.ds(cur * CH, CH)],
                         g_out.at[pl.ds(ebase + i * CH, CH)], semw.at[cur])
        return carry

    lax.fori_loop(0, NCHD, chunk, 0, unroll=False)
    for sl in (0, 1):
        ic = NCHD - 2 + sl
        pltpu.make_async_copy(hpg.at[pl.ds(sl * CH, CH)],
                              g_out.at[pl.ds(ebase + ic * CH, CH)],
                              semw.at[lax.rem(ic, 2)]).wait()


def _sc_edge(*args):
    if 'pe' not in _sc_cache:
        _sc_cache['pe'] = pl.kernel(
            _sc_edge_body,
            out_type=jax.ShapeDtypeStruct((EE, H), _f32),
            mesh=_get_mesh(),
            compiler_params=_sc_params,
            scratch_types=[
                pltpu.VMEM((NCHDP, CH), _i32),
                pltpu.VMEM((NCHDP, CH), _i32),
                pltpu.VMEM((2 * CH, H), _f32),
                pltpu.VMEM((2 * CH, H), _f32),
                pltpu.SemaphoreType.DMA((2,)),
                pltpu.SemaphoreType.DMA((2,)),
                pltpu.SemaphoreType.DMA((2,)),
            ],
        )
    return _sc_cache['pe'](*args)


def kernel(x_member, x_provider, edge_index_pm, edge_index_mp, edge_attr_pm, edge_attr_mp, params):
    ei_pm = edge_index_pm.astype(_i32)
    ei_mp = edge_index_mp.astype(_i32)
    def _pad_idx(flat, ew, nchp):
        a = flat.reshape(NW, ew)
        a = jnp.pad(a, ((0, 0), (0, nchp * CH - ew)))
        return a.reshape(NW * nchp, CH)

    src2d = _pad_idx(jnp.concatenate([ei_pm[0], ei_mp[0] + N]), EW, NCHP)
    dst2d = _pad_idx(jnp.concatenate([ei_pm[1], ei_mp[1]]), EW, NCHP)
    row2d = _pad_idx(ei_pm[0], EWD, NCHDP)
    col2d = _pad_idx(ei_pm[1], EWD, NCHDP)
    ea_cat = jnp.concatenate([edge_attr_pm, edge_attr_mp]).reshape(NW, EW, ED)
    ea_pad = jnp.pad(ea_cat, ((0, 0), (0, EWP - EW), (0, 0))).reshape(NW * EWP, ED)

    convs = params['convs']
    w_all = jnp.stack([
        jnp.stack([(cv[d]['W_edge'].T @ cv[d]['att_edge'][0])[None, :]
                   for d in ('pm', 'mp')])
        for cv in convs])
    eatt = _eatt(ea_pad, w_all)

    x_in = jnp.stack([x_member, x_provider])
    wp = jnp.stack([params['proj_member']['W'], params['proj_provider']['W']])
    bp = jnp.stack([params['proj_member']['b'], params['proj_provider']['b']])[:, None, :]
    x_state = _proj(x_in, wp, bp)

    for li, cv in enumerate(convs):
        w_src = jnp.stack([cv['pm']['W_src'], cv['mp']['W_src']])
        a_src = jnp.stack([cv['pm']['att_src'], cv['mp']['att_src']])
        w_dst_att = jnp.stack([(cv['pm']['att_dst'][0] @ cv['pm']['W_dst'])[None, :],
                               (cv['mp']['att_dst'][0] @ cv['mp']['W_dst'])[None, :]])
        w_edge = jnp.stack([cv['pm']['W_edge'], cv['mp']['W_edge']])
        xs_cat, s_src, s_dst = _prep(x_state, w_src, a_src, w_dst_att)
        ss_flat = jnp.pad(s_src.reshape(2 * N), (0, ROWS - 2 * N))
        sd_flat = jnp.pad(s_dst, ((0, 0), (0, 0), (0, RAC - N))).reshape(2 * RAC)
        den, num = _sc_pass1(src2d, dst2d, eatt[li], ss_flat, sd_flat)
        acc128, acc16 = _sc_pass2(src2d, dst2d, num, xs_cat, ea_pad)
        x_state = _combine(acc128, acc16, w_edge, x_state, den)

    wf = jnp.stack([params['final_member']['W'], params['final_provider']['W']])
    bf = jnp.stack([params['final_member']['b'], params['final_provider']['b']])[:, None, :]
    dm, dp, de = params['dec_member'], params['dec_provider'], params['dec_edge']
    w1 = jnp.stack([dm['W1'], dp['W1']])
    b1 = jnp.stack([dm['b1'], dp['b1']])[:, None, :]
    w2 = jnp.stack([dm['W2'], dp['W2']])
    b2 = jnp.stack([dm['b2'], dp['b2']])[:, None, :]
    w1e = jnp.stack([de['W1'][:, L:], de['W1'][:, :L]])
    b1e = jnp.stack([jnp.zeros_like(de['b1']), de['b1']])[:, None, :]
    z, xhat, hedge = _findec(x_state, wf, bf, w1, b1, w2, b2, w1e, b1e)

    g = _sc_edge(row2d, col2d, hedge[1], hedge[0])
    edge_hat = _edgemm(g, de['W2'], de['b2'][None, :])

    return xhat[0], xhat[1], z[0], z[1], edge_hat


# speedup vs baseline: 1.5951x; 1.5497x over previous
import functools

import jax
import jax.numpy as jnp
from jax import lax
from jax.experimental import pallas as pl
from jax.experimental.pallas import tpu as pltpu
from jax.experimental.pallas import tpu_sc as plsc

N = 5000
EE = 320000
TE = 2 * EE
H = 128
ED = 16
L = 64
NW = 32
CH = 80
EW = TE // NW
NCH = EW // CH
NCHP = 256
EWP = NCHP * CH
ROWS = 10240
RAC = 5120
RW = RAC // 16
EWD = EE // NW
NCHD = EWD // CH
NCHDP = 128

_f32 = jnp.float32
_i32 = jnp.int32


def _proj_body(x_ref, w_ref, b_ref, o_ref):
    y = lax.dot_general(x_ref[0], w_ref[0], (((1,), (1,)), ((), ())),
                        preferred_element_type=_f32) + b_ref[0, 0][None, :]
    o_ref[0] = jnp.where(y > 0, y, jnp.exp(jnp.minimum(y, 0.0)) - 1.0)


def _proj(x_stack, w_stack, b_stack):
    return pl.pallas_call(
        _proj_body,
        grid=(2,),
        in_specs=[
            pl.BlockSpec((1, N, H), lambda d: (d, 0, 0)),
            pl.BlockSpec((1, H, H), lambda d: (d, 0, 0)),
            pl.BlockSpec((1, 1, H), lambda d: (d, 0, 0)),
        ],
        out_specs=pl.BlockSpec((1, N, H), lambda d: (d, 0, 0)),
        out_shape=jax.ShapeDtypeStruct((2, N, H), _f32),
    )(x_stack, w_stack, b_stack)


_EB = 128


def _eatt_body(ea_ref, w_ref, o_ref):
    s = jnp.sum(ea_ref[...] * w_ref[0, 0, 0][None, :], axis=1)
    o_ref[0] = s.reshape(_EB, CH)


def _eatt(ea_pad, w_all):
    nb = NW * NCHP // _EB
    return pl.pallas_call(
        _eatt_body,
        grid=(2, nb),
        in_specs=[
            pl.BlockSpec((_EB * CH, ED), lambda l, i: (i, 0)),
            pl.BlockSpec((1, 1, 1, ED), lambda l, i: (l, i // (nb // 2), 0, 0)),
        ],
        out_specs=pl.BlockSpec((1, _EB, CH), lambda l, i: (l, i, 0)),
        out_shape=jax.ShapeDtypeStruct((2, NW * NCHP, CH), _f32),
    )(ea_pad, w_all)


def _prep_body(xs_ref, xd_ref, w_ref, asrc_ref, wdst_ref, xso_ref, ss_ref, sd_ref):
    xs = lax.dot_general(xs_ref[0], w_ref[0], (((1,), (1,)), ((), ())),
                         preferred_element_type=_f32)
    xso_ref[...] = xs
    ss_ref[0, 0] = jnp.sum(xs * asrc_ref[0, 0][None, :], axis=1)
    sd_ref[0, 0] = jnp.sum(xd_ref[0] * wdst_ref[0, 0][None, :], axis=1)


def _prep(x_state, w_src, a_src, w_dst_att):
    return pl.pallas_call(
        _prep_body,
        grid=(2,),
        in_specs=[
            pl.BlockSpec((1, N, H), lambda d: (1 - d, 0, 0)),
            pl.BlockSpec((1, N, H), lambda d: (d, 0, 0)),
            pl.BlockSpec((1, H, H), lambda d: (d, 0, 0)),
            pl.BlockSpec((1, 1, H), lambda d: (d, 0, 0)),
            pl.BlockSpec((1, 1, H), lambda d: (d, 0, 0)),
        ],
        out_specs=[
            pl.BlockSpec((N, H), lambda d: (d, 0)),
            pl.BlockSpec((1, 1, N), lambda d: (d, 0, 0)),
            pl.BlockSpec((1, 1, N), lambda d: (d, 0, 0)),
        ],
        out_shape=[
            jax.ShapeDtypeStruct((2 * N, H), _f32),
            jax.ShapeDtypeStruct((2, 1, N), _f32),
            jax.ShapeDtypeStruct((2, 1, N), _f32),
        ],
    )(x_state, x_state, w_src, a_src, w_dst_att)


def _combine_body(a128_ref, a16_ref, we_ref, xp_ref, den_ref, o_ref):
    a128 = a128_ref[0, :N]
    a16 = a16_ref[0, :N]
    inv = 1.0 / (den_ref[0, 0, :N] + 1e-16)
    y = (a128 + lax.dot_general(a16, we_ref[0], (((1,), (1,)), ((), ())),
                                preferred_element_type=_f32)) * inv[:, None] + xp_ref[0]
    o_ref[0] = jnp.where(y > 0, y, jnp.exp(jnp.minimum(y, 0.0)) - 1.0)


def _combine(acc128, acc16, w_edge, x_state, den):
    return pl.pallas_call(
        _combine_body,
        grid=(2,),
        in_specs=[
            pl.BlockSpec((1, RAC, H), lambda d: (d, 0, 0)),
            pl.BlockSpec((1, RAC, ED), lambda d: (d, 0, 0)),
            pl.BlockSpec((1, H, ED), lambda d: (d, 0, 0)),
            pl.BlockSpec((1, N, H), lambda d: (d, 0, 0)),
            pl.BlockSpec((1, 1, RAC), lambda d: (d, 0, 0)),
        ],
        out_specs=pl.BlockSpec((1, N, H), lambda d: (d, 0, 0)),
        out_shape=jax.ShapeDtypeStruct((2, N, H), _f32),
    )(acc128, acc16, w_edge, x_state, den.reshape(2, 1, RAC))


def _findec_body(x_ref, wf_ref, bf_ref, w1_ref, b1_ref, w2_ref, b2_ref,
                 w1e_ref, b1e_ref, z_ref, xh_ref, he_ref):
    z = lax.dot_general(x_ref[0], wf_ref[0], (((1,), (1,)), ((), ())),
                        preferred_element_type=_f32) + bf_ref[0, 0][None, :]
    z_ref[0] = z
    h = lax.dot_general(z, w1_ref[0], (((1,), (1,)), ((), ())),
                        preferred_element_type=_f32) + b1_ref[0, 0][None, :]
    h = jnp.maximum(h, 0.0)
    xh_ref[0] = lax.dot_general(h, w2_ref[0], (((1,), (1,)), ((), ())),
                                preferred_element_type=_f32) + b2_ref[0, 0][None, :]
    he_ref[0] = lax.dot_general(z, w1e_ref[0], (((1,), (1,)), ((), ())),
                                preferred_element_type=_f32) + b1e_ref[0, 0][None, :]


def _findec(x_state, wf, bf, w1, b1, w2, b2, w1e, b1e):
    return pl.pallas_call(
        _findec_body,
        grid=(2,),
        in_specs=[
            pl.BlockSpec((1, N, H), lambda d: (d, 0, 0)),
            pl.BlockSpec((1, L, H), lambda d: (d, 0, 0)),
            pl.BlockSpec((1, 1, L), lambda d: (d, 0, 0)),
            pl.BlockSpec((1, H, L), lambda d: (d, 0, 0)),
            pl.BlockSpec((1, 1, H), lambda d: (d, 0, 0)),
            pl.BlockSpec((1, H, H), lambda d: (d, 0, 0)),
            pl.BlockSpec((1, 1, H), lambda d: (d, 0, 0)),
            pl.BlockSpec((1, H, L), lambda d: (d, 0, 0)),
            pl.BlockSpec((1, 1, H), lambda d: (d, 0, 0)),
        ],
        out_specs=[
            pl.BlockSpec((1, N, L), lambda d: (d, 0, 0)),
            pl.BlockSpec((1, N, H), lambda d: (d, 0, 0)),
            pl.BlockSpec((1, N, H), lambda d: (d, 0, 0)),
        ],
        out_shape=[
            jax.ShapeDtypeStruct((2, N, L), _f32),
            jax.ShapeDtypeStruct((2, N, H), _f32),
            jax.ShapeDtypeStruct((2, N, H), _f32),
        ],
    )(x_state, wf, bf, w1, b1, w2, b2, w1e, b1e)


_GB = 10000


def _edgemm_body(g_ref, w_ref, b_ref, o_ref):
    o_ref[...] = lax.dot_general(g_ref[...], w_ref[...], (((1,), (1,)), ((), ())),
                                 preferred_element_type=_f32) + b_ref[0][None, :]


def _edgemm(g, w2e, b2e):
    return pl.pallas_call(
        _edgemm_body,
        grid=(EE // _GB,),
        in_specs=[
            pl.BlockSpec((_GB, H), lambda i: (i, 0)),
            pl.BlockSpec((ED, H), lambda i: (0, 0)),
            pl.BlockSpec((1, ED), lambda i: (0, 0)),
        ],
        out_specs=pl.BlockSpec((_GB, ED), lambda i: (i, 0)),
        out_shape=jax.ShapeDtypeStruct((EE, ED), _f32),
    )(g, w2e, b2e)


_sc_params = pltpu.CompilerParams(needs_layout_passes=False, use_tc_tiling_on_sc=False)
_sc_cache = {}


def _get_mesh():
    return plsc.VectorSubcoreMesh(core_axis_name="c", subcore_axis_name="s")


def _sc_pass1_body(src_hbm, dst_hbm, eatt_hbm, ssrc_hbm, sdst_hbm,
                   den_out, num_out,
                   srcb, dstb, eab, ssrcb, sdstb, numb, zb, den_sh, sem):
    c = lax.axis_index("c")
    s = lax.axis_index("s")
    wid = c * 16 + s
    rowbase = wid * NCHP
    pltpu.sync_copy(src_hbm.at[pl.ds(rowbase, NCHP)], srcb)
    pltpu.sync_copy(dst_hbm.at[pl.ds(rowbase, NCHP)], dstb)
    pltpu.sync_copy(eatt_hbm.at[pl.ds(rowbase, NCHP)], eab)
    pltpu.sync_copy(ssrc_hbm, ssrcb)
    pltpu.sync_copy(sdst_hbm, sdstb)
    zeros = jnp.zeros((16,), _f32)
    for j in range(RW // 16):
        zb[pl.ds(j * 16, 16)] = zeros
    pltpu.sync_copy(zb, den_sh.at[pl.ds(s * RW, RW)])
    plsc.subcore_barrier()
    doff = c * RAC

    def chunk(i, carry):
        for v in range(CH // 16):
            sidx = srcb[i, pl.ds(v * 16, 16)]
            didx = dstb[i, pl.ds(v * 16, 16)] + doff
            a = (plsc.load_gather(ssrcb, [sidx])
                 + plsc.load_gather(sdstb, [didx])
                 + eab[i, pl.ds(v * 16, 16)])
            a = jnp.where(a > 0, a, a * 0.01)
            numb[i, pl.ds(v * 16, 16)] = jnp.exp(a)
        pltpu.sync_copy(numb.at[i], den_sh.at[dstb.at[i]], add=True)
        return carry

    lax.fori_loop(0, NCH, chunk, 0, unroll=False)
    pltpu.sync_copy(numb, num_out.at[pl.ds(rowbase, NCHP)])
    plsc.subcore_barrier()
    pltpu.sync_copy(den_sh.at[pl.ds(s * RW, RW)],
                    den_out.at[pl.ds(c * RAC + s * RW, RW)])


def _sc_pass1(*args):
    if 'p1' not in _sc_cache:
        _sc_cache['p1'] = pl.kernel(
            _sc_pass1_body,
            out_type=[
                jax.ShapeDtypeStruct((2 * RAC,), _f32),
                jax.ShapeDtypeStruct((NW * NCHP, CH), _f32),
            ],
            mesh=_get_mesh(),
            compiler_params=_sc_params,
            scratch_types=[
                pltpu.VMEM((NCHP, CH), _i32),
                pltpu.VMEM((NCHP, CH), _i32),
                pltpu.VMEM((NCHP, CH), _f32),
                pltpu.VMEM((ROWS,), _f32),
                pltpu.VMEM((ROWS,), _f32),
                pltpu.VMEM((NCHP, CH), _f32),
                pltpu.VMEM((RW,), _f32),
                pltpu.VMEM_SHARED((RAC,), _f32),
                pltpu.SemaphoreType.DMA,
            ],
        )
    return _sc_cache['p1'](*args)


def _sc_pass2_body(src_hbm, dst_hbm, num_hbm, xs_hbm, ea_hbm,
                   a128_out, a16_out,
                   srcb, dstb, numb, xsg, eag, z128, z16,
                   a128_sh, a16_sh, semx, semsc, semse):
    c = lax.axis_index("c")
    s = lax.axis_index("s")
    wid = c * 16 + s
    rowbase = wid * NCHP
    ebase = wid * EWP
    pltpu.sync_copy(src_hbm.at[pl.ds(rowbase, NCHP)], srcb)
    pltpu.sync_copy(num_hbm.at[pl.ds(rowbase, NCHP)], numb)

    zeros = jnp.zeros((16,), _f32)
    for j in range(16):
        for v in range(H // 16):
            z128[j, pl.ds(v * 16, 16)] = zeros
        z16[j] = zeros

    def zloop(j, carry):
        pltpu.sync_copy(z128, a128_sh.at[pl.ds(s * RW + j * 16, 16)])
        pltpu.sync_copy(z16, a16_sh.at[pl.ds(s * RW + j * 16, 16)])
        return carry

    lax.fori_loop(0, RW // 16, zloop, 0, unroll=False)
    plsc.subcore_barrier()

    def chunk(i, carry):
        sl_i = lax.rem(i, 2)

        @pl.when(i < NCH)
        def _():
            @pl.when(i >= 2)
            def _():
                dr2 = lax.rem(i - 2, 8)
                pltpu.make_async_copy(xsg.at[pl.ds(sl_i * CH, CH)],
                                      a128_sh.at[dstb.at[dr2]], semsc.at[sl_i]).wait()
                pltpu.make_async_copy(eag.at[pl.ds(sl_i * CH, CH)],
                                      a16_sh.at[dstb.at[dr2]], semse.at[sl_i]).wait()

            pltpu.async_copy(xs_hbm.at[srcb.at[i]],
                             xsg.at[pl.ds(sl_i * CH, CH)], semx.at[sl_i])

        @pl.when(i > 0)
        def _():
            ip = i - 1
            sl_p = lax.rem(ip, 2)

            @pl.when(lax.rem(ip, 8) == 0)
            def _():
                pltpu.sync_copy(dst_hbm.at[pl.ds(rowbase + ip, 8)], dstb)

            pltpu.sync_copy(ea_hbm.at[pl.ds(ebase + ip * CH, CH)],
                            eag.at[pl.ds(sl_p * CH, CH)])
            pltpu.make_async_copy(xs_hbm.at[srcb.at[ip]],
                                  xsg.at[pl.ds(sl_p * CH, CH)], semx.at[sl_p]).wait()
            bi = jnp.full((16,), 0, _i32) + ip
            co = sl_p * CH

            @plsc.parallel_loop(0, CH, unroll=4)
            def _(j):
                bj = jnp.full((16,), 0, _i32) + j
                nv = plsc.load_gather(numb, [bi, bj])
                for k in range(H // 16):
                    xsg[co + j, pl.ds(k * 16, 16)] = xsg[co + j, pl.ds(k * 16, 16)] * nv
                eag[co + j] = eag[co + j] * nv
            dr = lax.rem(ip, 8)
            pltpu.async_copy(xsg.at[pl.ds(sl_p * CH, CH)],
                             a128_sh.at[dstb.at[dr]], semsc.at[sl_p], add=True)
            pltpu.async_copy(eag.at[pl.ds(sl_p * CH, CH)],
                             a16_sh.at[dstb.at[dr]], semse.at[sl_p], add=True)

        return carry

    lax.fori_loop(0, NCH + 1, chunk, 0, unroll=False)
    for sl in (0, 1):
        dr2 = (NCH - 2 + sl) % 8
        pltpu.make_async_copy(xsg.at[pl.ds(sl * CH, CH)],
                              a128_sh.at[dstb.at[dr2]], semsc.at[sl]).wait()
        pltpu.make_async_copy(eag.at[pl.ds(sl * CH, CH)],
                              a16_sh.at[dstb.at[dr2]], semse.at[sl]).wait()
    plsc.subcore_barrier()
    pltpu.sync_copy(a128_sh.at[pl.ds(s * RW, RW)], a128_out.at[c, pl.ds(s * RW, RW), :])
    pltpu.sync_copy(a16_sh.at[pl.ds(s * RW, RW)], a16_out.at[c, pl.ds(s * RW, RW), :])


def _sc_pass2(*args):
    if 'p2' not in _sc_cache:
        _sc_cache['p2'] = pl.kernel(
            _sc_pass2_body,
            out_type=[
                jax.ShapeDtypeStruct((2, RAC, H), _f32),
                jax.ShapeDtypeStruct((2, RAC, ED), _f32),
            ],
            mesh=_get_mesh(),
            compiler_params=_sc_params,
            scratch_types=[
                pltpu.VMEM((NCHP, CH), _i32),
                pltpu.VMEM((8, CH), _i32),
                pltpu.VMEM((NCHP, CH), _f32),
                pltpu.VMEM((2 * CH, H), _f32),
                pltpu.VMEM((2 * CH, ED), _f32),
                pltpu.VMEM((16, H), _f32),
                pltpu.VMEM((16, ED), _f32),
                pltpu.VMEM_SHARED((RAC, H), _f32),
                pltpu.VMEM_SHARED((RAC, ED), _f32),
                pltpu.SemaphoreType.DMA((2,)),
                pltpu.SemaphoreType.DMA((2,)),
                pltpu.SemaphoreType.DMA((2,)),
            ],
        )
    return _sc_cache['p2'](*args)


def _sc_edge_body(row_hbm, col_hbm, hp_hbm, hm_hbm, g_out,
                  rowb, colb, hpg, hmg, semp, semm, semw):
    c = lax.axis_index("c")
    s = lax.axis_index("s")
    wid = c * 16 + s
    rowbase = wid * NCHDP
    ebase = wid * EWD
    pltpu.sync_copy(row_hbm.at[pl.ds(rowbase, NCHDP)], rowb)
    pltpu.sync_copy(col_hbm.at[pl.ds(rowbase, NCHDP)], colb)

    pltpu.async_copy(hp_hbm.at[rowb.at[0]], hpg.at[pl.ds(0, CH)], semp.at[0])
    pltpu.async_copy(hm_hbm.at[colb.at[0]], hmg.at[pl.ds(0, CH)], semm.at[0])

    def chunk(i, carry):
        cur = lax.rem(i, 2)
        nxt = lax.rem(i + 1, 2)

        @pl.when(i + 1 < NCHD)
        def _():
            @pl.when(i >= 1)
            def _():
                pltpu.make_async_copy(hpg.at[pl.ds(nxt * CH, CH)],
                                      g_out.at[pl.ds(ebase + (i - 1) * CH, CH)],
                                      semw.at[nxt]).wait()

            pltpu.async_copy(hp_hbm.at[rowb.at[i + 1]],
                             hpg.at[pl.ds(nxt * CH, CH)], semp.at[nxt])
            pltpu.async_copy(hm_hbm.at[colb.at[i + 1]],
                             hmg.at[pl.ds(nxt * CH, CH)], semm.at[nxt])

        pltpu.make_async_copy(hp_hbm.at[rowb.at[i]],
                              hpg.at[pl.ds(cur * CH, CH)], semp.at[cur]).wait()
        pltpu.make_async_copy(hm_hbm.at[colb.at[i]],
                              hmg.at[pl.ds(cur * CH, CH)], semm.at[cur]).wait()
        co = cur * CH

        @plsc.parallel_loop(0, CH, unroll=4)
        def _(j):
            for k in range(H // 16):
                sl = pl.ds(k * 16, 16)
                hpg[co + j, sl] = jnp.maximum(hpg[co + j, sl] + hmg[co + j, sl], 0.0)
        pltpu.async_copy(hpg.at[pl.ds(cur * CH, CH)],
                         g_out.at[pl.ds(ebase + i * CH, CH)], semw.at[cur])
        return carry

    lax.fori_loop(0, NCHD, chunk, 0, unroll=False)
    for sl in (0, 1):
        ic = NCHD - 2 + sl
        pltpu.make_async_copy(hpg.at[pl.ds(sl * CH, CH)],
                              g_out.at[pl.ds(ebase + ic * CH, CH)],
                              semw.at[lax.rem(ic, 2)]).wait()


def _sc_edge(*args):
    if 'pe' not in _sc_cache:
        _sc_cache['pe'] = pl.kernel(
            _sc_edge_body,
            out_type=jax.ShapeDtypeStruct((EE, H), _f32),
            mesh=_get_mesh(),
            compiler_params=_sc_params,
            scratch_types=[
                pltpu.VMEM((NCHDP, CH), _i32),
                pltpu.VMEM((NCHDP, CH), _i32),
                pltpu.VMEM((2 * CH, H), _f32),
                pltpu.VMEM((2 * CH, H), _f32),
                pltpu.SemaphoreType.DMA((2,)),
                pltpu.SemaphoreType.DMA((2,)),
                pltpu.SemaphoreType.DMA((2,)),
            ],
        )
    return _sc_cache['pe'](*args)


def kernel(x_member, x_provider, edge_index_pm, edge_index_mp, edge_attr_pm, edge_attr_mp, params):
    ei_pm = edge_index_pm.astype(_i32)
    ei_mp = edge_index_mp.astype(_i32)
    def _pad_idx(flat, ew, nchp):
        a = flat.reshape(NW, ew)
        a = jnp.pad(a, ((0, 0), (0, nchp * CH - ew)))
        return a.reshape(NW * nchp, CH)

    src2d = _pad_idx(jnp.concatenate([ei_pm[0], ei_mp[0] + N]), EW, NCHP)
    dst2d = _pad_idx(jnp.concatenate([ei_pm[1], ei_mp[1]]), EW, NCHP)
    row2d = _pad_idx(ei_pm[0], EWD, NCHDP)
    col2d = _pad_idx(ei_pm[1], EWD, NCHDP)
    ea_cat = jnp.concatenate([edge_attr_pm, edge_attr_mp]).reshape(NW, EW, ED)
    ea_pad = jnp.pad(ea_cat, ((0, 0), (0, EWP - EW), (0, 0))).reshape(NW * EWP, ED)

    convs = params['convs']
    w_all = jnp.stack([
        jnp.stack([(cv[d]['W_edge'].T @ cv[d]['att_edge'][0])[None, :]
                   for d in ('pm', 'mp')])
        for cv in convs])
    eatt = _eatt(ea_pad, w_all)

    x_in = jnp.stack([x_member, x_provider])
    wp = jnp.stack([params['proj_member']['W'], params['proj_provider']['W']])
    bp = jnp.stack([params['proj_member']['b'], params['proj_provider']['b']])[:, None, :]
    x_state = _proj(x_in, wp, bp)

    for li, cv in enumerate(convs):
        w_src = jnp.stack([cv['pm']['W_src'], cv['mp']['W_src']])
        a_src = jnp.stack([cv['pm']['att_src'], cv['mp']['att_src']])
        w_dst_att = jnp.stack([(cv['pm']['att_dst'][0] @ cv['pm']['W_dst'])[None, :],
                               (cv['mp']['att_dst'][0] @ cv['mp']['W_dst'])[None, :]])
        w_edge = jnp.stack([cv['pm']['W_edge'], cv['mp']['W_edge']])
        xs_cat, s_src, s_dst = _prep(x_state, w_src, a_src, w_dst_att)
        ss_flat = jnp.pad(s_src.reshape(2 * N), (0, ROWS - 2 * N))
        sd_flat = jnp.pad(s_dst, ((0, 0), (0, 0), (0, RAC - N))).reshape(2 * RAC)
        den, num = _sc_pass1(src2d, dst2d, eatt[li], ss_flat, sd_flat)
        acc128, acc16 = _sc_pass2(src2d, dst2d, num, xs_cat, ea_pad)
        x_state = _combine(acc128, acc16, w_edge, x_state, den)

    wf = jnp.stack([params['final_member']['W'], params['final_provider']['W']])
    bf = jnp.stack([params['final_member']['b'], params['final_provider']['b']])[:, None, :]
    dm, dp, de = params['dec_member'], params['dec_provider'], params['dec_edge']
    w1 = jnp.stack([dm['W1'], dp['W1']])
    b1 = jnp.stack([dm['b1'], dp['b1']])[:, None, :]
    w2 = jnp.stack([dm['W2'], dp['W2']])
    b2 = jnp.stack([dm['b2'], dp['b2']])[:, None, :]
    w1e = jnp.stack([de['W1'][:, L:], de['W1'][:, :L]])
    b1e = jnp.stack([jnp.zeros_like(de['b1']), de['b1']])[:, None, :]
    z, xhat, hedge = _findec(x_state, wf, bf, w1, b1, w2, b2, w1e, b1e)

    g = _sc_edge(row2d, col2d, hedge[1], hedge[0])
    edge_hat = _edgemm(g, de['W2'], de['b2'][None, :])

    return xhat[0], xhat[1], z[0], z[1], edge_hat


# speedup vs baseline: 1.5963x; 1.0007x over previous
import functools

import jax
import jax.numpy as jnp
from jax import lax
from jax.experimental import pallas as pl
from jax.experimental.pallas import tpu as pltpu
from jax.experimental.pallas import tpu_sc as plsc

N = 5000
EE = 320000
TE = 2 * EE
H = 128
ED = 16
L = 64
NW = 32
CH = 80
EW = TE // NW
NCH = EW // CH
NCHP = 256
EWP = NCHP * CH
ROWS = 10240
RAC = 5120
RW = RAC // 16
EWD = EE // NW
NCHD = EWD // CH
NCHDP = 128

_f32 = jnp.float32
_i32 = jnp.int32


def _proj_body(x_ref, w_ref, b_ref, o_ref):
    y = lax.dot_general(x_ref[0], w_ref[0], (((1,), (1,)), ((), ())),
                        preferred_element_type=_f32) + b_ref[0, 0][None, :]
    o_ref[0] = jnp.where(y > 0, y, jnp.exp(jnp.minimum(y, 0.0)) - 1.0)


def _proj(x_stack, w_stack, b_stack):
    return pl.pallas_call(
        _proj_body,
        grid=(2,),
        in_specs=[
            pl.BlockSpec((1, N, H), lambda d: (d, 0, 0)),
            pl.BlockSpec((1, H, H), lambda d: (d, 0, 0)),
            pl.BlockSpec((1, 1, H), lambda d: (d, 0, 0)),
        ],
        out_specs=pl.BlockSpec((1, N, H), lambda d: (d, 0, 0)),
        out_shape=jax.ShapeDtypeStruct((2, N, H), _f32),
    )(x_stack, w_stack, b_stack)


_EB = 128


def _eatt_body(ea_ref, w_ref, o_ref):
    s = jnp.sum(ea_ref[...] * w_ref[0, 0, 0][None, :], axis=1)
    o_ref[0] = s.reshape(_EB, CH)


def _eatt(ea_pad, w_all):
    nb = NW * NCHP // _EB
    return pl.pallas_call(
        _eatt_body,
        grid=(2, nb),
        in_specs=[
            pl.BlockSpec((_EB * CH, ED), lambda l, i: (i, 0)),
            pl.BlockSpec((1, 1, 1, ED), lambda l, i: (l, i // (nb // 2), 0, 0)),
        ],
        out_specs=pl.BlockSpec((1, _EB, CH), lambda l, i: (l, i, 0)),
        out_shape=jax.ShapeDtypeStruct((2, NW * NCHP, CH), _f32),
    )(ea_pad, w_all)


def _prep_body(xs_ref, xd_ref, w_ref, asrc_ref, wdst_ref, xso_ref, ss_ref, sd_ref):
    xs = lax.dot_general(xs_ref[0], w_ref[0], (((1,), (1,)), ((), ())),
                         preferred_element_type=_f32)
    xso_ref[...] = xs
    ss_ref[0, 0] = jnp.sum(xs * asrc_ref[0, 0][None, :], axis=1)
    sd_ref[0, 0] = jnp.sum(xd_ref[0] * wdst_ref[0, 0][None, :], axis=1)


def _prep(x_state, w_src, a_src, w_dst_att):
    return pl.pallas_call(
        _prep_body,
        grid=(2,),
        in_specs=[
            pl.BlockSpec((1, N, H), lambda d: (1 - d, 0, 0)),
            pl.BlockSpec((1, N, H), lambda d: (d, 0, 0)),
            pl.BlockSpec((1, H, H), lambda d: (d, 0, 0)),
            pl.BlockSpec((1, 1, H), lambda d: (d, 0, 0)),
            pl.BlockSpec((1, 1, H), lambda d: (d, 0, 0)),
        ],
        out_specs=[
            pl.BlockSpec((N, H), lambda d: (d, 0)),
            pl.BlockSpec((1, 1, N), lambda d: (d, 0, 0)),
            pl.BlockSpec((1, 1, N), lambda d: (d, 0, 0)),
        ],
        out_shape=[
            jax.ShapeDtypeStruct((2 * N, H), _f32),
            jax.ShapeDtypeStruct((2, 1, N), _f32),
            jax.ShapeDtypeStruct((2, 1, N), _f32),
        ],
    )(x_state, x_state, w_src, a_src, w_dst_att)


def _combine_body(a128_ref, a16_ref, we_ref, xp_ref, den_ref, o_ref):
    a128 = a128_ref[0, :N]
    a16 = a16_ref[0, :N]
    inv = 1.0 / (den_ref[0, 0, :N] + 1e-16)
    y = (a128 + lax.dot_general(a16, we_ref[0], (((1,), (1,)), ((), ())),
                                preferred_element_type=_f32)) * inv[:, None] + xp_ref[0]
    o_ref[0] = jnp.where(y > 0, y, jnp.exp(jnp.minimum(y, 0.0)) - 1.0)


def _combine(acc128, acc16, w_edge, x_state, den):
    return pl.pallas_call(
        _combine_body,
        grid=(2,),
        in_specs=[
            pl.BlockSpec((1, RAC, H), lambda d: (d, 0, 0)),
            pl.BlockSpec((1, RAC, ED), lambda d: (d, 0, 0)),
            pl.BlockSpec((1, H, ED), lambda d: (d, 0, 0)),
            pl.BlockSpec((1, N, H), lambda d: (d, 0, 0)),
            pl.BlockSpec((1, 1, RAC), lambda d: (d, 0, 0)),
        ],
        out_specs=pl.BlockSpec((1, N, H), lambda d: (d, 0, 0)),
        out_shape=jax.ShapeDtypeStruct((2, N, H), _f32),
    )(acc128, acc16, w_edge, x_state, den.reshape(2, 1, RAC))


def _findec_body(x_ref, wf_ref, bf_ref, w1_ref, b1_ref, w2_ref, b2_ref,
                 w1e_ref, b1e_ref, z_ref, xh_ref, he_ref):
    z = lax.dot_general(x_ref[0], wf_ref[0], (((1,), (1,)), ((), ())),
                        preferred_element_type=_f32) + bf_ref[0, 0][None, :]
    z_ref[0] = z
    h = lax.dot_general(z, w1_ref[0], (((1,), (1,)), ((), ())),
                        preferred_element_type=_f32) + b1_ref[0, 0][None, :]
    h = jnp.maximum(h, 0.0)
    xh_ref[0] = lax.dot_general(h, w2_ref[0], (((1,), (1,)), ((), ())),
                                preferred_element_type=_f32) + b2_ref[0, 0][None, :]
    he_ref[0] = lax.dot_general(z, w1e_ref[0], (((1,), (1,)), ((), ())),
                                preferred_element_type=_f32) + b1e_ref[0, 0][None, :]


def _findec(x_state, wf, bf, w1, b1, w2, b2, w1e, b1e):
    return pl.pallas_call(
        _findec_body,
        grid=(2,),
        in_specs=[
            pl.BlockSpec((1, N, H), lambda d: (d, 0, 0)),
            pl.BlockSpec((1, L, H), lambda d: (d, 0, 0)),
            pl.BlockSpec((1, 1, L), lambda d: (d, 0, 0)),
            pl.BlockSpec((1, H, L), lambda d: (d, 0, 0)),
            pl.BlockSpec((1, 1, H), lambda d: (d, 0, 0)),
            pl.BlockSpec((1, H, H), lambda d: (d, 0, 0)),
            pl.BlockSpec((1, 1, H), lambda d: (d, 0, 0)),
            pl.BlockSpec((1, H, L), lambda d: (d, 0, 0)),
            pl.BlockSpec((1, 1, H), lambda d: (d, 0, 0)),
        ],
        out_specs=[
            pl.BlockSpec((1, N, L), lambda d: (d, 0, 0)),
            pl.BlockSpec((1, N, H), lambda d: (d, 0, 0)),
            pl.BlockSpec((1, N, H), lambda d: (d, 0, 0)),
        ],
        out_shape=[
            jax.ShapeDtypeStruct((2, N, L), _f32),
            jax.ShapeDtypeStruct((2, N, H), _f32),
            jax.ShapeDtypeStruct((2, N, H), _f32),
        ],
    )(x_state, wf, bf, w1, b1, w2, b2, w1e, b1e)


_GB = 10000


def _edgemm_body(g_ref, w_ref, b_ref, o_ref):
    o_ref[...] = lax.dot_general(g_ref[...], w_ref[...], (((1,), (1,)), ((), ())),
                                 preferred_element_type=_f32) + b_ref[0][None, :]


def _edgemm(g, w2e, b2e):
    return pl.pallas_call(
        _edgemm_body,
        grid=(EE // _GB,),
        in_specs=[
            pl.BlockSpec((_GB, H), lambda i: (i, 0)),
            pl.BlockSpec((ED, H), lambda i: (0, 0)),
            pl.BlockSpec((1, ED), lambda i: (0, 0)),
        ],
        out_specs=pl.BlockSpec((_GB, ED), lambda i: (i, 0)),
        out_shape=jax.ShapeDtypeStruct((EE, ED), _f32),
    )(g, w2e, b2e)


_sc_params = pltpu.CompilerParams(needs_layout_passes=False, use_tc_tiling_on_sc=False)
_sc_cache = {}


def _get_mesh():
    return plsc.VectorSubcoreMesh(core_axis_name="c", subcore_axis_name="s")


def _sc_pass1_body(src_hbm, dst_hbm, eatt_hbm, ssrc_hbm, sdst_hbm,
                   den_out, num_out,
                   srcb, dstb, eab, ssrcb, sdstb, numb, zb, den_sh, sem):
    c = lax.axis_index("c")
    s = lax.axis_index("s")
    wid = c * 16 + s
    rowbase = wid * NCHP
    pltpu.sync_copy(src_hbm.at[pl.ds(rowbase, NCHP)], srcb)
    pltpu.sync_copy(dst_hbm.at[pl.ds(rowbase, NCHP)], dstb)
    pltpu.sync_copy(eatt_hbm.at[pl.ds(rowbase, NCHP)], eab)
    pltpu.sync_copy(ssrc_hbm, ssrcb)
    pltpu.sync_copy(sdst_hbm, sdstb)
    zeros = jnp.zeros((16,), _f32)
    for j in range(RW // 16):
        zb[pl.ds(j * 16, 16)] = zeros
    pltpu.sync_copy(zb, den_sh.at[pl.ds(s * RW, RW)])
    plsc.subcore_barrier()
    doff = c * RAC

    def chunk(i, carry):
        for v in range(CH // 16):
            sidx = srcb[i, pl.ds(v * 16, 16)]
            didx = dstb[i, pl.ds(v * 16, 16)] + doff
            a = (plsc.load_gather(ssrcb, [sidx])
                 + plsc.load_gather(sdstb, [didx])
                 + eab[i, pl.ds(v * 16, 16)])
            a = jnp.where(a > 0, a, a * 0.01)
            numb[i, pl.ds(v * 16, 16)] = jnp.exp(a)
        pltpu.sync_copy(numb.at[i], den_sh.at[dstb.at[i]], add=True)
        return carry

    lax.fori_loop(0, NCH, chunk, 0, unroll=False)
    pltpu.sync_copy(numb, num_out.at[pl.ds(rowbase, NCHP)])
    plsc.subcore_barrier()
    pltpu.sync_copy(den_sh.at[pl.ds(s * RW, RW)],
                    den_out.at[pl.ds(c * RAC + s * RW, RW)])


def _sc_pass1(*args):
    if 'p1' not in _sc_cache:
        _sc_cache['p1'] = pl.kernel(
            _sc_pass1_body,
            out_type=[
                jax.ShapeDtypeStruct((2 * RAC,), _f32),
                jax.ShapeDtypeStruct((NW * NCHP, CH), _f32),
            ],
            mesh=_get_mesh(),
            compiler_params=_sc_params,
            scratch_types=[
                pltpu.VMEM((NCHP, CH), _i32),
                pltpu.VMEM((NCHP, CH), _i32),
                pltpu.VMEM((NCHP, CH), _f32),
                pltpu.VMEM((ROWS,), _f32),
                pltpu.VMEM((ROWS,), _f32),
                pltpu.VMEM((NCHP, CH), _f32),
                pltpu.VMEM((RW,), _f32),
                pltpu.VMEM_SHARED((RAC,), _f32),
                pltpu.SemaphoreType.DMA,
            ],
        )
    return _sc_cache['p1'](*args)


def _sc_pass2_body(src_hbm, dst_hbm, num_hbm, xs_hbm, ea_hbm,
                   a128_out, a16_out,
                   srcb, dstb, numb, xsg, eag, z128, z16,
                   a128_sh, a16_sh, semx, semsc, semse):
    c = lax.axis_index("c")
    s = lax.axis_index("s")
    wid = c * 16 + s
    rowbase = wid * NCHP
    ebase = wid * EWP
    pltpu.sync_copy(src_hbm.at[pl.ds(rowbase, NCHP)], srcb)
    pltpu.sync_copy(num_hbm.at[pl.ds(rowbase, NCHP)], numb)

    zeros = jnp.zeros((16,), _f32)
    for j in range(16):
        for v in range(H // 16):
            z128[j, pl.ds(v * 16, 16)] = zeros
        z16[j] = zeros

    def zloop(j, carry):
        pltpu.sync_copy(z128, a128_sh.at[pl.ds(s * RW + j * 16, 16)])
        pltpu.sync_copy(z16, a16_sh.at[pl.ds(s * RW + j * 16, 16)])
        return carry

    lax.fori_loop(0, RW // 16, zloop, 0, unroll=False)
    plsc.subcore_barrier()

    def chunk(i, carry):
        sl_i = lax.rem(i, 2)

        @pl.when(i < NCH)
        def _():
            @pl.when(i >= 2)
            def _():
                dr2 = lax.rem(i - 2, 8)
                pltpu.make_async_copy(xsg.at[pl.ds(sl_i * CH, CH)],
                                      a128_sh.at[dstb.at[dr2]], semsc.at[sl_i]).wait()
                pltpu.make_async_copy(eag.at[pl.ds(sl_i * CH, CH)],
                                      a16_sh.at[dstb.at[dr2]], semse.at[sl_i]).wait()

            pltpu.async_copy(xs_hbm.at[srcb.at[i]],
                             xsg.at[pl.ds(sl_i * CH, CH)], semx.at[sl_i])

        @pl.when(i > 0)
        def _():
            ip = i - 1
            sl_p = lax.rem(ip, 2)

            @pl.when(lax.rem(ip, 8) == 0)
            def _():
                pltpu.sync_copy(dst_hbm.at[pl.ds(rowbase + ip, 8)], dstb)

            pltpu.sync_copy(ea_hbm.at[pl.ds(ebase + ip * CH, CH)],
                            eag.at[pl.ds(sl_p * CH, CH)])
            pltpu.make_async_copy(xs_hbm.at[srcb.at[ip]],
                                  xsg.at[pl.ds(sl_p * CH, CH)], semx.at[sl_p]).wait()
            bi = jnp.full((16,), 0, _i32) + ip
            co = sl_p * CH

            @plsc.parallel_loop(0, CH, unroll=8)
            def _(j):
                bj = jnp.full((16,), 0, _i32) + j
                nv = plsc.load_gather(numb, [bi, bj])
                for k in range(H // 16):
                    xsg[co + j, pl.ds(k * 16, 16)] = xsg[co + j, pl.ds(k * 16, 16)] * nv
                eag[co + j] = eag[co + j] * nv
            dr = lax.rem(ip, 8)
            pltpu.async_copy(xsg.at[pl.ds(sl_p * CH, CH)],
                             a128_sh.at[dstb.at[dr]], semsc.at[sl_p], add=True)
            pltpu.async_copy(eag.at[pl.ds(sl_p * CH, CH)],
                             a16_sh.at[dstb.at[dr]], semse.at[sl_p], add=True)

        return carry

    lax.fori_loop(0, NCH + 1, chunk, 0, unroll=False)
    for sl in (0, 1):
        dr2 = (NCH - 2 + sl) % 8
        pltpu.make_async_copy(xsg.at[pl.ds(sl * CH, CH)],
                              a128_sh.at[dstb.at[dr2]], semsc.at[sl]).wait()
        pltpu.make_async_copy(eag.at[pl.ds(sl * CH, CH)],
                              a16_sh.at[dstb.at[dr2]], semse.at[sl]).wait()
    plsc.subcore_barrier()
    pltpu.sync_copy(a128_sh.at[pl.ds(s * RW, RW)], a128_out.at[c, pl.ds(s * RW, RW), :])
    pltpu.sync_copy(a16_sh.at[pl.ds(s * RW, RW)], a16_out.at[c, pl.ds(s * RW, RW), :])


def _sc_pass2(*args):
    if 'p2' not in _sc_cache:
        _sc_cache['p2'] = pl.kernel(
            _sc_pass2_body,
            out_type=[
                jax.ShapeDtypeStruct((2, RAC, H), _f32),
                jax.ShapeDtypeStruct((2, RAC, ED), _f32),
            ],
            mesh=_get_mesh(),
            compiler_params=_sc_params,
            scratch_types=[
                pltpu.VMEM((NCHP, CH), _i32),
                pltpu.VMEM((8, CH), _i32),
                pltpu.VMEM((NCHP, CH), _f32),
                pltpu.VMEM((2 * CH, H), _f32),
                pltpu.VMEM((2 * CH, ED), _f32),
                pltpu.VMEM((16, H), _f32),
                pltpu.VMEM((16, ED), _f32),
                pltpu.VMEM_SHARED((RAC, H), _f32),
                pltpu.VMEM_SHARED((RAC, ED), _f32),
                pltpu.SemaphoreType.DMA((2,)),
                pltpu.SemaphoreType.DMA((2,)),
                pltpu.SemaphoreType.DMA((2,)),
            ],
        )
    return _sc_cache['p2'](*args)


def _sc_edge_body(row_hbm, col_hbm, hp_hbm, hm_hbm, g_out,
                  rowb, colb, hpg, hmg, semp, semm, semw):
    c = lax.axis_index("c")
    s = lax.axis_index("s")
    wid = c * 16 + s
    rowbase = wid * NCHDP
    ebase = wid * EWD
    pltpu.sync_copy(row_hbm.at[pl.ds(rowbase, NCHDP)], rowb)
    pltpu.sync_copy(col_hbm.at[pl.ds(rowbase, NCHDP)], colb)

    pltpu.async_copy(hp_hbm.at[rowb.at[0]], hpg.at[pl.ds(0, CH)], semp.at[0])
    pltpu.async_copy(hm_hbm.at[colb.at[0]], hmg.at[pl.ds(0, CH)], semm.at[0])

    def chunk(i, carry):
        cur = lax.rem(i, 2)
        nxt = lax.rem(i + 1, 2)

        @pl.when(i + 1 < NCHD)
        def _():
            @pl.when(i >= 1)
            def _():
                pltpu.make_async_copy(hpg.at[pl.ds(nxt * CH, CH)],
                                      g_out.at[pl.ds(ebase + (i - 1) * CH, CH)],
                                      semw.at[nxt]).wait()

            pltpu.async_copy(hp_hbm.at[rowb.at[i + 1]],
                             hpg.at[pl.ds(nxt * CH, CH)], semp.at[nxt])
            pltpu.async_copy(hm_hbm.at[colb.at[i + 1]],
                             hmg.at[pl.ds(nxt * CH, CH)], semm.at[nxt])

        pltpu.make_async_copy(hp_hbm.at[rowb.at[i]],
                              hpg.at[pl.ds(cur * CH, CH)], semp.at[cur]).wait()
        pltpu.make_async_copy(hm_hbm.at[colb.at[i]],
                              hmg.at[pl.ds(cur * CH, CH)], semm.at[cur]).wait()
        co = cur * CH

        @plsc.parallel_loop(0, CH, unroll=8)
        def _(j):
            for k in range(H // 16):
                sl = pl.ds(k * 16, 16)
                hpg[co + j, sl] = jnp.maximum(hpg[co + j, sl] + hmg[co + j, sl], 0.0)
        pltpu.async_copy(hpg.at[pl.ds(cur * CH, CH)],
                         g_out.at[pl.ds(ebase + i * CH, CH)], semw.at[cur])
        return carry

    lax.fori_loop(0, NCHD, chunk, 0, unroll=False)
    for sl in (0, 1):
        ic = NCHD - 2 + sl
        pltpu.make_async_copy(hpg.at[pl.ds(sl * CH, CH)],
                              g_out.at[pl.ds(ebase + ic * CH, CH)],
                              semw.at[lax.rem(ic, 2)]).wait()


def _sc_edge(*args):
    if 'pe' not in _sc_cache:
        _sc_cache['pe'] = pl.kernel(
            _sc_edge_body,
            out_type=jax.ShapeDtypeStruct((EE, H), _f32),
            mesh=_get_mesh(),
            compiler_params=_sc_params,
            scratch_types=[
                pltpu.VMEM((NCHDP, CH), _i32),
                pltpu.VMEM((NCHDP, CH), _i32),
                pltpu.VMEM((2 * CH, H), _f32),
                pltpu.VMEM((2 * CH, H), _f32),
                pltpu.SemaphoreType.DMA((2,)),
                pltpu.SemaphoreType.DMA((2,)),
                pltpu.SemaphoreType.DMA((2,)),
            ],
        )
    return _sc_cache['pe'](*args)


def kernel(x_member, x_provider, edge_index_pm, edge_index_mp, edge_attr_pm, edge_attr_mp, params):
    ei_pm = edge_index_pm.astype(_i32)
    ei_mp = edge_index_mp.astype(_i32)
    def _pad_idx(flat, ew, nchp):
        a = flat.reshape(NW, ew)
        a = jnp.pad(a, ((0, 0), (0, nchp * CH - ew)))
        return a.reshape(NW * nchp, CH)

    src2d = _pad_idx(jnp.concatenate([ei_pm[0], ei_mp[0] + N]), EW, NCHP)
    dst2d = _pad_idx(jnp.concatenate([ei_pm[1], ei_mp[1]]), EW, NCHP)
    row2d = _pad_idx(ei_pm[0], EWD, NCHDP)
    col2d = _pad_idx(ei_pm[1], EWD, NCHDP)
    ea_cat = jnp.concatenate([edge_attr_pm, edge_attr_mp]).reshape(NW, EW, ED)
    ea_pad = jnp.pad(ea_cat, ((0, 0), (0, EWP - EW), (0, 0))).reshape(NW * EWP, ED)

    convs = params['convs']
    w_all = jnp.stack([
        jnp.stack([(cv[d]['W_edge'].T @ cv[d]['att_edge'][0])[None, :]
                   for d in ('pm', 'mp')])
        for cv in convs])
    eatt = _eatt(ea_pad, w_all)

    x_in = jnp.stack([x_member, x_provider])
    wp = jnp.stack([params['proj_member']['W'], params['proj_provider']['W']])
    bp = jnp.stack([params['proj_member']['b'], params['proj_provider']['b']])[:, None, :]
    x_state = _proj(x_in, wp, bp)

    for li, cv in enumerate(convs):
        w_src = jnp.stack([cv['pm']['W_src'], cv['mp']['W_src']])
        a_src = jnp.stack([cv['pm']['att_src'], cv['mp']['att_src']])
        w_dst_att = jnp.stack([(cv['pm']['att_dst'][0] @ cv['pm']['W_dst'])[None, :],
                               (cv['mp']['att_dst'][0] @ cv['mp']['W_dst'])[None, :]])
        w_edge = jnp.stack([cv['pm']['W_edge'], cv['mp']['W_edge']])
        xs_cat, s_src, s_dst = _prep(x_state, w_src, a_src, w_dst_att)
        ss_flat = jnp.pad(s_src.reshape(2 * N), (0, ROWS - 2 * N))
        sd_flat = jnp.pad(s_dst, ((0, 0), (0, 0), (0, RAC - N))).reshape(2 * RAC)
        den, num = _sc_pass1(src2d, dst2d, eatt[li], ss_flat, sd_flat)
        acc128, acc16 = _sc_pass2(src2d, dst2d, num, xs_cat, ea_pad)
        x_state = _combine(acc128, acc16, w_edge, x_state, den)

    wf = jnp.stack([params['final_member']['W'], params['final_provider']['W']])
    bf = jnp.stack([params['final_member']['b'], params['final_provider']['b']])[:, None, :]
    dm, dp, de = params['dec_member'], params['dec_provider'], params['dec_edge']
    w1 = jnp.stack([dm['W1'], dp['W1']])
    b1 = jnp.stack([dm['b1'], dp['b1']])[:, None, :]
    w2 = jnp.stack([dm['W2'], dp['W2']])
    b2 = jnp.stack([dm['b2'], dp['b2']])[:, None, :]
    w1e = jnp.stack([de['W1'][:, L:], de['W1'][:, :L]])
    b1e = jnp.stack([jnp.zeros_like(de['b1']), de['b1']])[:, None, :]
    z, xhat, hedge = _findec(x_state, wf, bf, w1, b1, w2, b2, w1e, b1e)

    g = _sc_edge(row2d, col2d, hedge[1], hedge[0])
    edge_hat = _edgemm(g, de['W2'], de['b2'][None, :])

    return xhat[0], xhat[1], z[0], z[1], edge_hat


# speedup vs baseline: 1.8294x; 1.1460x over previous
import functools

import jax
import jax.numpy as jnp
from jax import lax
from jax.experimental import pallas as pl
from jax.experimental.pallas import tpu as pltpu
from jax.experimental.pallas import tpu_sc as plsc

N = 5000
EE = 320000
TE = 2 * EE
H = 128
ED = 16
L = 64
NW = 32
CH = 80
EW = TE // NW
NCH = EW // CH
NCHP = 256
EWP = NCHP * CH
ROWS = 10240
RAC = 5120
RW = RAC // 16
EWD = EE // NW
NCHD = EWD // CH
NCHDP = 128

_f32 = jnp.float32
_i32 = jnp.int32


def _proj_body(x_ref, w_ref, b_ref, o_ref):
    y = lax.dot_general(x_ref[0], w_ref[0], (((1,), (1,)), ((), ())),
                        preferred_element_type=_f32) + b_ref[0, 0][None, :]
    o_ref[0] = jnp.where(y > 0, y, jnp.exp(jnp.minimum(y, 0.0)) - 1.0)


def _proj(x_stack, w_stack, b_stack):
    return pl.pallas_call(
        _proj_body,
        grid=(2,),
        in_specs=[
            pl.BlockSpec((1, N, H), lambda d: (d, 0, 0)),
            pl.BlockSpec((1, H, H), lambda d: (d, 0, 0)),
            pl.BlockSpec((1, 1, H), lambda d: (d, 0, 0)),
        ],
        out_specs=pl.BlockSpec((1, N, H), lambda d: (d, 0, 0)),
        out_shape=jax.ShapeDtypeStruct((2, N, H), _f32),
    )(x_stack, w_stack, b_stack)


_EB = 128


def _eatt_body(ea_ref, w_ref, o_ref):
    s = jnp.sum(ea_ref[...] * w_ref[0, 0, 0][None, :], axis=1)
    o_ref[0] = s.reshape(_EB, CH)


def _eatt(ea_pad, w_all):
    nb = NW * NCHP // _EB
    return pl.pallas_call(
        _eatt_body,
        grid=(2, nb),
        in_specs=[
            pl.BlockSpec((_EB * CH, ED), lambda l, i: (i, 0)),
            pl.BlockSpec((1, 1, 1, ED), lambda l, i: (l, i // (nb // 2), 0, 0)),
        ],
        out_specs=pl.BlockSpec((1, _EB, CH), lambda l, i: (l, i, 0)),
        out_shape=jax.ShapeDtypeStruct((2, NW * NCHP, CH), _f32),
    )(ea_pad, w_all)


def _prep_body(xs_ref, xd_ref, w_ref, asrc_ref, wdst_ref, xso_ref, ss_ref, sd_ref):
    xs = lax.dot_general(xs_ref[0], w_ref[0], (((1,), (1,)), ((), ())),
                         preferred_element_type=_f32)
    xso_ref[...] = xs
    ss_ref[0, 0] = jnp.sum(xs * asrc_ref[0, 0][None, :], axis=1)
    sd_ref[0, 0] = jnp.sum(xd_ref[0] * wdst_ref[0, 0][None, :], axis=1)


def _prep(x_state, w_src, a_src, w_dst_att):
    return pl.pallas_call(
        _prep_body,
        grid=(2,),
        in_specs=[
            pl.BlockSpec((1, N, H), lambda d: (1 - d, 0, 0)),
            pl.BlockSpec((1, N, H), lambda d: (d, 0, 0)),
            pl.BlockSpec((1, H, H), lambda d: (d, 0, 0)),
            pl.BlockSpec((1, 1, H), lambda d: (d, 0, 0)),
            pl.BlockSpec((1, 1, H), lambda d: (d, 0, 0)),
        ],
        out_specs=[
            pl.BlockSpec((N, H), lambda d: (d, 0)),
            pl.BlockSpec((1, 1, N), lambda d: (d, 0, 0)),
            pl.BlockSpec((1, 1, N), lambda d: (d, 0, 0)),
        ],
        out_shape=[
            jax.ShapeDtypeStruct((2 * N, H), _f32),
            jax.ShapeDtypeStruct((2, 1, N), _f32),
            jax.ShapeDtypeStruct((2, 1, N), _f32),
        ],
    )(x_state, x_state, w_src, a_src, w_dst_att)


def _combine_body(a128_ref, a16_ref, we_ref, xp_ref, den_ref, o_ref):
    a128 = a128_ref[0, :N]
    a16 = a16_ref[0, :N]
    inv = 1.0 / (den_ref[0, 0, :N] + 1e-16)
    y = (a128 + lax.dot_general(a16, we_ref[0], (((1,), (1,)), ((), ())),
                                preferred_element_type=_f32)) * inv[:, None] + xp_ref[0]
    o_ref[0] = jnp.where(y > 0, y, jnp.exp(jnp.minimum(y, 0.0)) - 1.0)


def _combine(acc128, acc16, w_edge, x_state, den):
    return pl.pallas_call(
        _combine_body,
        grid=(2,),
        in_specs=[
            pl.BlockSpec((1, RAC, H), lambda d: (d, 0, 0)),
            pl.BlockSpec((1, RAC, ED), lambda d: (d, 0, 0)),
            pl.BlockSpec((1, H, ED), lambda d: (d, 0, 0)),
            pl.BlockSpec((1, N, H), lambda d: (d, 0, 0)),
            pl.BlockSpec((1, 1, RAC), lambda d: (d, 0, 0)),
        ],
        out_specs=pl.BlockSpec((1, N, H), lambda d: (d, 0, 0)),
        out_shape=jax.ShapeDtypeStruct((2, N, H), _f32),
    )(acc128, acc16, w_edge, x_state, den.reshape(2, 1, RAC))


def _findec_body(x_ref, wf_ref, bf_ref, w1_ref, b1_ref, w2_ref, b2_ref,
                 w1e_ref, b1e_ref, z_ref, xh_ref, he_ref):
    z = lax.dot_general(x_ref[0], wf_ref[0], (((1,), (1,)), ((), ())),
                        preferred_element_type=_f32) + bf_ref[0, 0][None, :]
    z_ref[0] = z
    h = lax.dot_general(z, w1_ref[0], (((1,), (1,)), ((), ())),
                        preferred_element_type=_f32) + b1_ref[0, 0][None, :]
    h = jnp.maximum(h, 0.0)
    xh_ref[0] = lax.dot_general(h, w2_ref[0], (((1,), (1,)), ((), ())),
                                preferred_element_type=_f32) + b2_ref[0, 0][None, :]
    he_ref[0] = lax.dot_general(z, w1e_ref[0], (((1,), (1,)), ((), ())),
                                preferred_element_type=_f32) + b1e_ref[0, 0][None, :]


def _findec(x_state, wf, bf, w1, b1, w2, b2, w1e, b1e):
    return pl.pallas_call(
        _findec_body,
        grid=(2,),
        in_specs=[
            pl.BlockSpec((1, N, H), lambda d: (d, 0, 0)),
            pl.BlockSpec((1, L, H), lambda d: (d, 0, 0)),
            pl.BlockSpec((1, 1, L), lambda d: (d, 0, 0)),
            pl.BlockSpec((1, H, L), lambda d: (d, 0, 0)),
            pl.BlockSpec((1, 1, H), lambda d: (d, 0, 0)),
            pl.BlockSpec((1, H, H), lambda d: (d, 0, 0)),
            pl.BlockSpec((1, 1, H), lambda d: (d, 0, 0)),
            pl.BlockSpec((1, H, L), lambda d: (d, 0, 0)),
            pl.BlockSpec((1, 1, H), lambda d: (d, 0, 0)),
        ],
        out_specs=[
            pl.BlockSpec((1, N, L), lambda d: (d, 0, 0)),
            pl.BlockSpec((1, N, H), lambda d: (d, 0, 0)),
            pl.BlockSpec((1, N, H), lambda d: (d, 0, 0)),
        ],
        out_shape=[
            jax.ShapeDtypeStruct((2, N, L), _f32),
            jax.ShapeDtypeStruct((2, N, H), _f32),
            jax.ShapeDtypeStruct((2, N, H), _f32),
        ],
    )(x_state, wf, bf, w1, b1, w2, b2, w1e, b1e)


_GB = 10000


def _edgemm_body(g_ref, w_ref, b_ref, o_ref):
    o_ref[...] = lax.dot_general(g_ref[...], w_ref[...], (((1,), (1,)), ((), ())),
                                 preferred_element_type=_f32) + b_ref[0][None, :]


def _edgemm(g, w2e, b2e):
    return pl.pallas_call(
        _edgemm_body,
        grid=(EE // _GB,),
        in_specs=[
            pl.BlockSpec((_GB, H), lambda i: (i, 0)),
            pl.BlockSpec((ED, H), lambda i: (0, 0)),
            pl.BlockSpec((1, ED), lambda i: (0, 0)),
        ],
        out_specs=pl.BlockSpec((_GB, ED), lambda i: (i, 0)),
        out_shape=jax.ShapeDtypeStruct((EE, ED), _f32),
    )(g, w2e, b2e)


_sc_params = pltpu.CompilerParams(needs_layout_passes=False, use_tc_tiling_on_sc=False)
_sc_cache = {}


def _get_mesh():
    return plsc.VectorSubcoreMesh(core_axis_name="c", subcore_axis_name="s")


def _sc_pass1_body(src_hbm, dst_hbm, eatt_hbm, ssrc_hbm, sdst_hbm,
                   den_out, num_out,
                   srcb, dstb, eab, ssrcb, sdstb, numb, zb, den_sh, sem):
    c = lax.axis_index("c")
    s = lax.axis_index("s")
    wid = c * 16 + s
    rowbase = wid * NCHP
    pltpu.sync_copy(src_hbm.at[pl.ds(rowbase, NCHP)], srcb)
    pltpu.sync_copy(dst_hbm.at[pl.ds(rowbase, NCHP)], dstb)
    pltpu.sync_copy(eatt_hbm.at[pl.ds(rowbase, NCHP)], eab)
    pltpu.sync_copy(ssrc_hbm, ssrcb)
    pltpu.sync_copy(sdst_hbm, sdstb)
    zeros = jnp.zeros((16,), _f32)
    for j in range(RW // 16):
        zb[pl.ds(j * 16, 16)] = zeros
    pltpu.sync_copy(zb, den_sh.at[pl.ds(s * RW, RW)])
    plsc.subcore_barrier()
    doff = c * RAC

    def chunk(i, carry):
        for v in range(CH // 16):
            sidx = srcb[i, pl.ds(v * 16, 16)]
            didx = dstb[i, pl.ds(v * 16, 16)] + doff
            a = (plsc.load_gather(ssrcb, [sidx])
                 + plsc.load_gather(sdstb, [didx])
                 + eab[i, pl.ds(v * 16, 16)])
            a = jnp.where(a > 0, a, a * 0.01)
            numb[i, pl.ds(v * 16, 16)] = jnp.exp(a)
        pltpu.sync_copy(numb.at[i], den_sh.at[dstb.at[i]], add=True)
        return carry

    lax.fori_loop(0, NCH, chunk, 0, unroll=False)
    pltpu.sync_copy(numb, num_out.at[pl.ds(rowbase, NCHP)])
    plsc.subcore_barrier()
    pltpu.sync_copy(den_sh.at[pl.ds(s * RW, RW)],
                    den_out.at[pl.ds(c * RAC + s * RW, RW)])


def _sc_pass1(*args):
    if 'p1' not in _sc_cache:
        _sc_cache['p1'] = pl.kernel(
            _sc_pass1_body,
            out_type=[
                jax.ShapeDtypeStruct((2 * RAC,), _f32),
                jax.ShapeDtypeStruct((NW * NCHP, CH), _f32),
            ],
            mesh=_get_mesh(),
            compiler_params=_sc_params,
            scratch_types=[
                pltpu.VMEM((NCHP, CH), _i32),
                pltpu.VMEM((NCHP, CH), _i32),
                pltpu.VMEM((NCHP, CH), _f32),
                pltpu.VMEM((ROWS,), _f32),
                pltpu.VMEM((ROWS,), _f32),
                pltpu.VMEM((NCHP, CH), _f32),
                pltpu.VMEM((RW,), _f32),
                pltpu.VMEM_SHARED((RAC,), _f32),
                pltpu.SemaphoreType.DMA,
            ],
        )
    return _sc_cache['p1'](*args)


def _sc_pass2_body(src_hbm, dst_hbm, num_hbm, xs_hbm, ea_hbm,
                   a128_out, a16_out,
                   srcb, dstb, numb, xsg, eag, z128, z16,
                   a128_sh, a16_sh, semx, semg, semsc, semse):
    c = lax.axis_index("c")
    s = lax.axis_index("s")
    wid = c * 16 + s
    rowbase = wid * NCHP
    ebase = wid * EWP
    pltpu.sync_copy(src_hbm.at[pl.ds(rowbase, NCHP)], srcb)
    pltpu.sync_copy(num_hbm.at[pl.ds(rowbase, NCHP)], numb)

    zeros = jnp.zeros((16,), _f32)
    for j in range(16):
        for v in range(H // 16):
            z128[j, pl.ds(v * 16, 16)] = zeros
        z16[j] = zeros

    def zloop(j, carry):
        pltpu.sync_copy(z128, a128_sh.at[pl.ds(s * RW + j * 16, 16)])
        pltpu.sync_copy(z16, a16_sh.at[pl.ds(s * RW + j * 16, 16)])
        return carry

    lax.fori_loop(0, RW // 16, zloop, 0, unroll=False)
    plsc.subcore_barrier()

    def chunk(i, carry):
        sl_i = lax.rem(i, 2)

        @pl.when(i < NCH)
        def _():
            @pl.when(i >= 2)
            def _():
                dr2 = lax.rem(i - 2, 8)
                pltpu.make_async_copy(xsg.at[pl.ds(sl_i * CH, CH)],
                                      a128_sh.at[dstb.at[dr2]], semsc.at[sl_i]).wait()
                pltpu.make_async_copy(eag.at[pl.ds(sl_i * CH, CH)],
                                      a16_sh.at[dstb.at[dr2]], semse.at[sl_i]).wait()

            pltpu.async_copy(xs_hbm.at[srcb.at[i]],
                             xsg.at[pl.ds(sl_i * CH, CH)], semx.at[sl_i])
            pltpu.async_copy(ea_hbm.at[pl.ds(ebase + i * CH, CH)],
                             eag.at[pl.ds(sl_i * CH, CH)], semg.at[sl_i])

        @pl.when(i > 0)
        def _():
            ip = i - 1
            sl_p = lax.rem(ip, 2)

            @pl.when(lax.rem(ip, 8) == 0)
            def _():
                pltpu.sync_copy(dst_hbm.at[pl.ds(rowbase + ip, 8)], dstb)

            pltpu.make_async_copy(ea_hbm.at[pl.ds(ebase + ip * CH, CH)],
                                  eag.at[pl.ds(sl_p * CH, CH)], semg.at[sl_p]).wait()
            pltpu.make_async_copy(xs_hbm.at[srcb.at[ip]],
                                  xsg.at[pl.ds(sl_p * CH, CH)], semx.at[sl_p]).wait()
            bi = jnp.full((16,), 0, _i32) + ip
            co = sl_p * CH

            @plsc.parallel_loop(0, CH, unroll=8)
            def _(j):
                bj = jnp.full((16,), 0, _i32) + j
                nv = plsc.load_gather(numb, [bi, bj])
                for k in range(H // 16):
                    xsg[co + j, pl.ds(k * 16, 16)] = xsg[co + j, pl.ds(k * 16, 16)] * nv
                eag[co + j] = eag[co + j] * nv
            dr = lax.rem(ip, 8)
            pltpu.async_copy(xsg.at[pl.ds(sl_p * CH, CH)],
                             a128_sh.at[dstb.at[dr]], semsc.at[sl_p], add=True)
            pltpu.async_copy(eag.at[pl.ds(sl_p * CH, CH)],
                             a16_sh.at[dstb.at[dr]], semse.at[sl_p], add=True)

        return carry

    lax.fori_loop(0, NCH + 1, chunk, 0, unroll=False)
    for sl in (0, 1):
        dr2 = (NCH - 2 + sl) % 8
        pltpu.make_async_copy(xsg.at[pl.ds(sl * CH, CH)],
                              a128_sh.at[dstb.at[dr2]], semsc.at[sl]).wait()
        pltpu.make_async_copy(eag.at[pl.ds(sl * CH, CH)],
                              a16_sh.at[dstb.at[dr2]], semse.at[sl]).wait()
    plsc.subcore_barrier()
    pltpu.sync_copy(a128_sh.at[pl.ds(s * RW, RW)], a128_out.at[c, pl.ds(s * RW, RW), :])
    pltpu.sync_copy(a16_sh.at[pl.ds(s * RW, RW)], a16_out.at[c, pl.ds(s * RW, RW), :])


def _sc_pass2(*args):
    if 'p2' not in _sc_cache:
        _sc_cache['p2'] = pl.kernel(
            _sc_pass2_body,
            out_type=[
                jax.ShapeDtypeStruct((2, RAC, H), _f32),
                jax.ShapeDtypeStruct((2, RAC, ED), _f32),
            ],
            mesh=_get_mesh(),
            compiler_params=_sc_params,
            scratch_types=[
                pltpu.VMEM((NCHP, CH), _i32),
                pltpu.VMEM((8, CH), _i32),
                pltpu.VMEM((NCHP, CH), _f32),
                pltpu.VMEM((2 * CH, H), _f32),
                pltpu.VMEM((2 * CH, ED), _f32),
                pltpu.VMEM((16, H), _f32),
                pltpu.VMEM((16, ED), _f32),
                pltpu.VMEM_SHARED((RAC, H), _f32),
                pltpu.VMEM_SHARED((RAC, ED), _f32),
                pltpu.SemaphoreType.DMA((2,)),
                pltpu.SemaphoreType.DMA((2,)),
                pltpu.SemaphoreType.DMA((2,)),
                pltpu.SemaphoreType.DMA((2,)),
            ],
        )
    return _sc_cache['p2'](*args)


def _sc_edge_body(row_hbm, col_hbm, hp_hbm, hm_hbm, g_out,
                  rowb, colb, hpg, hmg, semp, semm, semw):
    c = lax.axis_index("c")
    s = lax.axis_index("s")
    wid = c * 16 + s
    rowbase = wid * NCHDP
    ebase = wid * EWD
    pltpu.sync_copy(row_hbm.at[pl.ds(rowbase, NCHDP)], rowb)
    pltpu.sync_copy(col_hbm.at[pl.ds(rowbase, NCHDP)], colb)

    pltpu.async_copy(hp_hbm.at[rowb.at[0]], hpg.at[pl.ds(0, CH)], semp.at[0])
    pltpu.async_copy(hm_hbm.at[colb.at[0]], hmg.at[pl.ds(0, CH)], semm.at[0])

    def chunk(i, carry):
        cur = lax.rem(i, 2)
        nxt = lax.rem(i + 1, 2)

        @pl.when(i + 1 < NCHD)
        def _():
            @pl.when(i >= 1)
            def _():
                pltpu.make_async_copy(hpg.at[pl.ds(nxt * CH, CH)],
                                      g_out.at[pl.ds(ebase + (i - 1) * CH, CH)],
                                      semw.at[nxt]).wait()

            pltpu.async_copy(hp_hbm.at[rowb.at[i + 1]],
                             hpg.at[pl.ds(nxt * CH, CH)], semp.at[nxt])
            pltpu.async_copy(hm_hbm.at[colb.at[i + 1]],
                             hmg.at[pl.ds(nxt * CH, CH)], semm.at[nxt])

        pltpu.make_async_copy(hp_hbm.at[rowb.at[i]],
                              hpg.at[pl.ds(cur * CH, CH)], semp.at[cur]).wait()
        pltpu.make_async_copy(hm_hbm.at[colb.at[i]],
                              hmg.at[pl.ds(cur * CH, CH)], semm.at[cur]).wait()
        co = cur * CH

        @plsc.parallel_loop(0, CH, unroll=8)
        def _(j):
            for k in range(H // 16):
                sl = pl.ds(k * 16, 16)
                hpg[co + j, sl] = jnp.maximum(hpg[co + j, sl] + hmg[co + j, sl], 0.0)
        pltpu.async_copy(hpg.at[pl.ds(cur * CH, CH)],
                         g_out.at[pl.ds(ebase + i * CH, CH)], semw.at[cur])
        return carry

    lax.fori_loop(0, NCHD, chunk, 0, unroll=False)
    for sl in (0, 1):
        ic = NCHD - 2 + sl
        pltpu.make_async_copy(hpg.at[pl.ds(sl * CH, CH)],
                              g_out.at[pl.ds(ebase + ic * CH, CH)],
                              semw.at[lax.rem(ic, 2)]).wait()


def _sc_edge(*args):
    if 'pe' not in _sc_cache:
        _sc_cache['pe'] = pl.kernel(
            _sc_edge_body,
            out_type=jax.ShapeDtypeStruct((EE, H), _f32),
            mesh=_get_mesh(),
            compiler_params=_sc_params,
            scratch_types=[
                pltpu.VMEM((NCHDP, CH), _i32),
                pltpu.VMEM((NCHDP, CH), _i32),
                pltpu.VMEM((2 * CH, H), _f32),
                pltpu.VMEM((2 * CH, H), _f32),
                pltpu.SemaphoreType.DMA((2,)),
                pltpu.SemaphoreType.DMA((2,)),
                pltpu.SemaphoreType.DMA((2,)),
            ],
        )
    return _sc_cache['pe'](*args)


def kernel(x_member, x_provider, edge_index_pm, edge_index_mp, edge_attr_pm, edge_attr_mp, params):
    ei_pm = edge_index_pm.astype(_i32)
    ei_mp = edge_index_mp.astype(_i32)
    def _pad_idx(flat, ew, nchp):
        a = flat.reshape(NW, ew)
        a = jnp.pad(a, ((0, 0), (0, nchp * CH - ew)))
        return a.reshape(NW * nchp, CH)

    src2d = _pad_idx(jnp.concatenate([ei_pm[0], ei_mp[0] + N]), EW, NCHP)
    dst2d = _pad_idx(jnp.concatenate([ei_pm[1], ei_mp[1]]), EW, NCHP)
    row2d = _pad_idx(ei_pm[0], EWD, NCHDP)
    col2d = _pad_idx(ei_pm[1], EWD, NCHDP)
    ea_cat = jnp.concatenate([edge_attr_pm, edge_attr_mp]).reshape(NW, EW, ED)
    ea_pad = jnp.pad(ea_cat, ((0, 0), (0, EWP - EW), (0, 0))).reshape(NW * EWP, ED)

    convs = params['convs']
    w_all = jnp.stack([
        jnp.stack([(cv[d]['W_edge'].T @ cv[d]['att_edge'][0])[None, :]
                   for d in ('pm', 'mp')])
        for cv in convs])
    eatt = _eatt(ea_pad, w_all)

    x_in = jnp.stack([x_member, x_provider])
    wp = jnp.stack([params['proj_member']['W'], params['proj_provider']['W']])
    bp = jnp.stack([params['proj_member']['b'], params['proj_provider']['b']])[:, None, :]
    x_state = _proj(x_in, wp, bp)

    for li, cv in enumerate(convs):
        w_src = jnp.stack([cv['pm']['W_src'], cv['mp']['W_src']])
        a_src = jnp.stack([cv['pm']['att_src'], cv['mp']['att_src']])
        w_dst_att = jnp.stack([(cv['pm']['att_dst'][0] @ cv['pm']['W_dst'])[None, :],
                               (cv['mp']['att_dst'][0] @ cv['mp']['W_dst'])[None, :]])
        w_edge = jnp.stack([cv['pm']['W_edge'], cv['mp']['W_edge']])
        xs_cat, s_src, s_dst = _prep(x_state, w_src, a_src, w_dst_att)
        ss_flat = jnp.pad(s_src.reshape(2 * N), (0, ROWS - 2 * N))
        sd_flat = jnp.pad(s_dst, ((0, 0), (0, 0), (0, RAC - N))).reshape(2 * RAC)
        den, num = _sc_pass1(src2d, dst2d, eatt[li], ss_flat, sd_flat)
        acc128, acc16 = _sc_pass2(src2d, dst2d, num, xs_cat, ea_pad)
        x_state = _combine(acc128, acc16, w_edge, x_state, den)

    wf = jnp.stack([params['final_member']['W'], params['final_provider']['W']])
    bf = jnp.stack([params['final_member']['b'], params['final_provider']['b']])[:, None, :]
    dm, dp, de = params['dec_member'], params['dec_provider'], params['dec_edge']
    w1 = jnp.stack([dm['W1'], dp['W1']])
    b1 = jnp.stack([dm['b1'], dp['b1']])[:, None, :]
    w2 = jnp.stack([dm['W2'], dp['W2']])
    b2 = jnp.stack([dm['b2'], dp['b2']])[:, None, :]
    w1e = jnp.stack([de['W1'][:, L:], de['W1'][:, :L]])
    b1e = jnp.stack([jnp.zeros_like(de['b1']), de['b1']])[:, None, :]
    z, xhat, hedge = _findec(x_state, wf, bf, w1, b1, w2, b2, w1e, b1e)

    g = _sc_edge(row2d, col2d, hedge[1], hedge[0])
    edge_hat = _edgemm(g, de['W2'], de['b2'][None, :])

    return xhat[0], xhat[1], z[0], z[1], edge_hat
